# Initial kernel scaffold; baseline (speedup 1.0000x reference)
import jax
import jax.numpy as jnp
from jax.experimental import pallas as pl


def kernel(x, edge_index, edge_attrs, stem_types, stems, stems_batch, batch, x_slices, frag_emb_w, stem_emb_w, bond_emb_w, conv_root_w, conv_bias, gru_w_ih, gru_w_hh, gru_b_ih, gru_b_hh, f2e_w1, f2e_b1, f2e_w2, f2e_b2, s2p_w1, s2p_b1, s2p_w2, s2p_b2, s2p_w3, s2p_b3, g2p_w1, g2p_b1, g2p_w2, g2p_b2):
    raise NotImplementedError("write your pallas kernel here")


# speedup vs baseline: 3.2349x; 3.2349x over previous
import functools

import jax
import jax.numpy as jnp
from jax import lax
from jax.experimental import pallas as pl
from jax.experimental.pallas import tpu as pltpu
from jax.experimental.pallas import tpu_sc as plsc

EMB = 32
NC = 2
NS = 16
NW = NC * NS
CH = 128
NUM_CONV_STEPS = 4

f32 = jnp.float32
i32 = jnp.int32


def _ceil_to(v, m):
    return (v + m - 1) // m * m


def _lrelu(v):
    return jnp.where(v >= 0, v, 0.01 * v)


def _tc_prep_body(frag_ref, w1t_ref, b1_ref, w2t_ref, b2_ref, stem_ref,
                  w1bt_ref, t1_ref, t2_ref):
    z = _lrelu(jnp.dot(frag_ref[...], w1t_ref[...],
                       preferred_element_type=f32) + b1_ref[...])
    t1_ref[...] = jnp.dot(z, w2t_ref[...], preferred_element_type=f32) + b2_ref[...]
    t2_ref[...] = jnp.dot(stem_ref[...], w1bt_ref[...], preferred_element_type=f32)


def _tc_gru_body(agg_ref, h_ref, degp_ref, wrt_ref,
                 wir_ref, wiz_ref, win_ref, whr_ref, whz_ref, whn_ref,
                 cb_ref, bir_ref, biz_ref, bin_ref, bhr_ref, bhz_ref, bhn_ref,
                 out_ref):
    deg = jnp.sum(degp_ref[...], axis=1)
    deginv = 1.0 / jnp.maximum(deg, 1.0)
    hb = h_ref[...]
    agg = (agg_ref[0] + agg_ref[1]) * deginv[:, None]
    conv = agg + jnp.dot(hb, wrt_ref[...], preferred_element_type=f32) + cb_ref[...]
    m = _lrelu(conv)
    gir = jnp.dot(m, wir_ref[...], preferred_element_type=f32) + bir_ref[...]
    giz = jnp.dot(m, wiz_ref[...], preferred_element_type=f32) + biz_ref[...]
    gin = jnp.dot(m, win_ref[...], preferred_element_type=f32) + bin_ref[...]
    ghr = jnp.dot(hb, whr_ref[...], preferred_element_type=f32) + bhr_ref[...]
    ghz = jnp.dot(hb, whz_ref[...], preferred_element_type=f32) + bhz_ref[...]
    ghn = jnp.dot(hb, whn_ref[...], preferred_element_type=f32) + bhn_ref[...]
    r = jax.nn.sigmoid(gir + ghr)
    z = jax.nn.sigmoid(giz + ghz)
    n = jnp.tanh(gin + r * ghn)
    out_ref[...] = (1.0 - z) * n + z * hb


def _tc_stems_body(hs_ref, st2_ref, w1at_ref, b1_ref, w2t_ref, b2_ref,
                   w3t_ref, b3_ref, out_ref):
    sp = _lrelu(jnp.dot(hs_ref[...], w1at_ref[...], preferred_element_type=f32)
                + st2_ref[...] + b1_ref[...])
    sp = _lrelu(jnp.dot(sp, w2t_ref[...], preferred_element_type=f32) + b2_ref[...])
    out_ref[...] = jnp.dot(sp, w3t_ref[...], preferred_element_type=f32) + b3_ref[...]


def _tc_pool_body(h3_ref, w1t_ref, b1_ref, w2t_ref, b2_ref, out_ref):
    pooled = jnp.mean(h3_ref[...], axis=1)
    mp = _lrelu(jnp.dot(pooled, w1t_ref[...], preferred_element_type=f32) + b1_ref[...])
    out_ref[...] = jnp.dot(mp, w2t_ref[...], preferred_element_type=f32) + b2_ref[...]


def _mesh():
    return plsc.VectorSubcoreMesh(core_axis_name="c", subcore_axis_name="s")


_SC_PARAMS = pltpu.CompilerParams(needs_layout_passes=False,
                                  use_tc_tiling_on_sc=False)


def _wid():
    return lax.axis_index("s") * NC + lax.axis_index("c")


def _make_sc_prep(n_pad, s_pad, n_deg, e_pad):
    nch_n = n_pad // (NW * CH)
    nch_s = s_pad // (NW * CH)
    nch_e = e_pad // (NW * CH)
    et = e_pad // NW

    @functools.partial(
        pl.kernel,
        out_type=(
            jax.ShapeDtypeStruct((n_pad, EMB), f32),
            jax.ShapeDtypeStruct((s_pad, EMB), f32),
            jax.ShapeDtypeStruct((s_pad,), i32),
            jax.ShapeDtypeStruct((NW, (n_deg + 16) // 16, 16), f32),
        ),
        mesh=_mesh(),
        compiler_params=_SC_PARAMS,
        scratch_types=[
            pltpu.VMEM((CH,), i32),
            pltpu.VMEM((CH, EMB), f32),
            pltpu.VMEM((CH,), i32),
            pltpu.VMEM((CH,), i32),
            pltpu.VMEM((16, 16), f32),
            pltpu.VMEM(((n_deg + 16) // 16, 16), f32),
            pltpu.SemaphoreType.DMA,
        ],
    )
    def sc_prep(t1_hbm, t2_hbm, xg_hbm, stg_hbm, sb_hbm, s0_hbm, xsl_hbm,
                dst_hbm, zdeg_hbm,
                h0_hbm, st2_hbm, sidx_hbm, degpart_hbm,
                idxb, rowb, s0b, sidxb, xslv, degv, sem):
        w = _wid()
        ii = lax.iota(i32, 16)
        for k in range(nch_n):
            base = w * (nch_n * CH) + k * CH
            pltpu.sync_copy(xg_hbm.at[pl.ds(base, CH)], idxb)
            pltpu.async_copy(t1_hbm.at[idxb], rowb, sem).wait()
            pltpu.sync_copy(rowb, h0_hbm.at[pl.ds(base, CH)])
        for k in range(nch_s):
            base = w * (nch_s * CH) + k * CH
            pltpu.sync_copy(stg_hbm.at[pl.ds(base, CH)], idxb)
            pltpu.async_copy(t2_hbm.at[idxb], rowb, sem).wait()
            pltpu.sync_copy(rowb, st2_hbm.at[pl.ds(base, CH)])
        pltpu.sync_copy(xsl_hbm, xslv)
        for k in range(nch_s):
            base = w * (nch_s * CH) + k * CH
            pltpu.sync_copy(sb_hbm.at[pl.ds(base, CH)], idxb)
            pltpu.sync_copy(s0_hbm.at[pl.ds(base, CH)], s0b)
            for g in range(CH // 16):
                sb = idxb[pl.ds(g * 16, 16)]
                sv = plsc.bitcast(plsc.load_gather(
                    xslv, [lax.shift_right_logical(sb, 4),
                           lax.bitwise_and(sb, 15)]), i32)
                sidxb[pl.ds(g * 16, 16)] = sv + s0b[pl.ds(g * 16, 16)]
            pltpu.sync_copy(sidxb, sidx_hbm.at[pl.ds(base, CH)])
        pltpu.sync_copy(zdeg_hbm, degv)
        ones16 = jnp.full((16,), 1.0, dtype=f32)
        for k in range(nch_e):
            base = w * et + k * CH
            pltpu.sync_copy(dst_hbm.at[pl.ds(base, CH)], idxb)
            for g in range(CH // 16):
                dvec = idxb[pl.ds(g * 16, 16)]
                plsc.addupdate_scatter(
                    degv, [lax.shift_right_logical(dvec, 4),
                           lax.bitwise_and(dvec, 15)], ones16)
        pltpu.sync_copy(degv, degpart_hbm.at[w])

    return sc_prep


def _make_sc_edge(n, n_agg, e_pad, n_bond):
    et = e_pad // NW
    nch_e = et // CH
    zrows = n_agg // NS
    orows = n // NS

    @functools.partial(
        pl.kernel,
        out_type=jax.ShapeDtypeStruct((NC, n, EMB), f32),
        mesh=_mesh(),
        compiler_params=_SC_PARAMS,
        scratch_types=[
            pltpu.VMEM((CH,), i32),
            pltpu.VMEM((CH,), i32),
            pltpu.VMEM((CH,), i32),
            pltpu.VMEM((CH,), i32),
            pltpu.VMEM((CH, EMB), f32),
            pltpu.VMEM((CH, EMB), f32),
            pltpu.VMEM((n_bond, EMB), f32),
            pltpu.VMEM_SHARED((n_agg, EMB), f32),
            pltpu.SemaphoreType.DMA,
        ],
    )
    def sc_edge(h_hbm, src_hbm, dst_hbm, e0_hbm, e1_hbm, bond_hbm, z_hbm,
                agg_hbm,
                srcb, dstb, e0b, e1b, gbuf, msg, tabv, aggs, sem):
        cid = lax.axis_index("c")
        sid = lax.axis_index("s")
        w = sid * NC + cid
        ii = lax.iota(i32, 16)
        pltpu.sync_copy(z_hbm.at[pl.ds(sid * zrows, zrows)],
                        aggs.at[pl.ds(sid * zrows, zrows)])
        pltpu.sync_copy(bond_hbm, tabv)
        plsc.subcore_barrier()

        def chunk(k, carry):
            base = w * et + k * CH
            pltpu.sync_copy(src_hbm.at[pl.ds(base, CH)], srcb)
            pltpu.sync_copy(dst_hbm.at[pl.ds(base, CH)], dstb)
            pltpu.sync_copy(e0_hbm.at[pl.ds(base, CH)], e0b)
            pltpu.sync_copy(e1_hbm.at[pl.ds(base, CH)], e1b)
            pltpu.async_copy(h_hbm.at[srcb], gbuf, sem).wait()
            for g in range(CH // 16):
                gi = ii + (g * 16)
                ev0 = e0b[pl.ds(g * 16, 16)]
                ev1 = e1b[pl.ds(g * 16, 16)]
                acc = [jnp.zeros((16,), f32) for _ in range(4)]
                for c in range(EMB):
                    cv = jnp.full((16,), c, dtype=i32)
                    gc = plsc.load_gather(gbuf, [gi, cv])
                    ac = plsc.load_gather(tabv, [ev0, cv])
                    acc[c & 3] = acc[c & 3] + gc * ac
                s = (acc[0] + acc[1]) + (acc[2] + acc[3])
                for c in range(EMB):
                    cv = jnp.full((16,), c, dtype=i32)
                    bc = plsc.load_gather(tabv, [ev1, cv])
                    plsc.store_scatter(msg, [gi, cv], s * bc)
            pltpu.sync_copy(msg, aggs.at[dstb], add=True)
            return carry

        lax.fori_loop(0, nch_e, chunk, 0)
        plsc.subcore_barrier()
        pltpu.sync_copy(aggs.at[pl.ds(sid * orows, orows)],
                        agg_hbm.at[cid, pl.ds(sid * orows, orows)])

    return sc_edge


def _make_sc_hgather(n, s_pad):
    nch_s = s_pad // (NW * CH)

    @functools.partial(
        pl.kernel,
        out_type=jax.ShapeDtypeStruct((s_pad, EMB), f32),
        mesh=_mesh(),
        compiler_params=_SC_PARAMS,
        scratch_types=[
            pltpu.VMEM((CH,), i32),
            pltpu.VMEM((CH, EMB), f32),
            pltpu.SemaphoreType.DMA,
        ],
    )
    def sc_hgather(h_hbm, sidx_hbm, out_hbm, idxb, rowb, sem):
        w = _wid()
        for k in range(nch_s):
            base = w * (nch_s * CH) + k * CH
            pltpu.sync_copy(sidx_hbm.at[pl.ds(base, CH)], idxb)
            pltpu.async_copy(h_hbm.at[idxb], rowb, sem).wait()
            pltpu.sync_copy(rowb, out_hbm.at[pl.ds(base, CH)])

    return sc_hgather


def kernel(x, edge_index, edge_attrs, stem_types, stems, stems_batch, batch,
           x_slices, frag_emb_w, stem_emb_w, bond_emb_w, conv_root_w,
           conv_bias, gru_w_ih, gru_w_hh, gru_b_ih, gru_b_hh, f2e_w1, f2e_b1,
           f2e_w2, f2e_b2, s2p_w1, s2p_b1, s2p_w2, s2p_b2, s2p_w3, s2p_b3,
           g2p_w1, g2p_b1, g2p_w2, g2p_b2):
    n = x.shape[0]
    e = edge_index.shape[1]
    s = stem_types.shape[0]
    bn = x_slices.shape[0] - 1
    n_bond = bond_emb_w.shape[0]
    npg = n // bn

    e_pad = _ceil_to(e, NW * CH)
    n_pad = _ceil_to(n, NW * CH)
    s_pad = _ceil_to(s, NW * CH)
    n_agg = _ceil_to(n + 16, NS)

    srcp = jnp.concatenate([edge_index[0], jnp.zeros((e_pad - e,), i32)])
    dstp = jnp.concatenate([edge_index[1], jnp.full((e_pad - e,), n, i32)])
    e0p = jnp.concatenate([edge_attrs[:, 0], jnp.zeros((e_pad - e,), i32)])
    e1p = jnp.concatenate([edge_attrs[:, 1], jnp.zeros((e_pad - e,), i32)])
    xgp = jnp.concatenate([x, jnp.zeros((n_pad - n,), i32)])
    stgp = jnp.concatenate([stem_types, jnp.zeros((s_pad - s,), i32)])
    sbp = jnp.concatenate([stems_batch, jnp.zeros((s_pad - s,), i32)])
    s0p = jnp.concatenate([stems[:, 0], jnp.zeros((s_pad - s,), i32)])
    xslp = jnp.concatenate(
        [x_slices, jnp.zeros((256 - x_slices.shape[0],), i32)]).reshape(16, 16)
    xslp = lax.bitcast_convert_type(xslp, f32)
    zeros_agg = jnp.zeros((n_agg, EMB), f32)
    zeros_deg = jnp.zeros(((n + 16) // 16, 16), f32)

    w1at = s2p_w1[:, :EMB].T
    w1bt = s2p_w1[:, EMB:].T
    wirt, wizt, wint = (gru_w_ih[0:EMB].T, gru_w_ih[EMB:2 * EMB].T,
                        gru_w_ih[2 * EMB:].T)
    whrt, whzt, whnt = (gru_w_hh[0:EMB].T, gru_w_hh[EMB:2 * EMB].T,
                        gru_w_hh[2 * EMB:].T)
    bir, biz, bin_ = (gru_b_ih[0:EMB].reshape(1, EMB),
                      gru_b_ih[EMB:2 * EMB].reshape(1, EMB),
                      gru_b_ih[2 * EMB:].reshape(1, EMB))
    bhr, bhz, bhn = (gru_b_hh[0:EMB].reshape(1, EMB),
                     gru_b_hh[EMB:2 * EMB].reshape(1, EMB),
                     gru_b_hh[2 * EMB:].reshape(1, EMB))

    nf = frag_emb_w.shape[0]
    nst = stem_emb_w.shape[0]
    t1, t2 = pl.pallas_call(
        _tc_prep_body,
        out_shape=(jax.ShapeDtypeStruct((nf, EMB), f32),
                   jax.ShapeDtypeStruct((nst, EMB), f32)),
    )(frag_emb_w, f2e_w1.T, f2e_b1.reshape(1, EMB), f2e_w2.T,
      f2e_b2.reshape(1, EMB), stem_emb_w, w1bt)

    sc_prep = _make_sc_prep(n_pad, s_pad, n, e_pad)
    h0p, st2p, sidxp, degpart = sc_prep(
        t1, t2, xgp, stgp, sbp, s0p, xslp, dstp, zeros_deg)
    h = h0p[:n]

    sc_edge = _make_sc_edge(n, n_agg, e_pad, n_bond)
    rb = 1000
    grid = n // rb
    gru_call = pl.pallas_call(
        _tc_gru_body,
        grid=(grid,),
        in_specs=[
            pl.BlockSpec((NC, rb, EMB), lambda i: (0, i, 0)),
            pl.BlockSpec((rb, EMB), lambda i: (i, 0)),
            pl.BlockSpec((rb, NW), lambda i: (i, 0)),
        ] + [pl.BlockSpec((EMB, EMB), lambda i: (0, 0))] * 7
          + [pl.BlockSpec((1, EMB), lambda i: (0, 0))] * 7,
        out_specs=pl.BlockSpec((rb, EMB), lambda i: (i, 0)),
        out_shape=jax.ShapeDtypeStruct((n, EMB), f32),
    )
    degpt = degpart.reshape(NW, -1)[:, :n].T
    for _ in range(NUM_CONV_STEPS):
        aggp = sc_edge(h, srcp, dstp, e0p, e1p, bond_emb_w, zeros_agg)
        h = gru_call(aggp, h, degpt, conv_root_w.T, wirt, wizt, wint,
                     whrt, whzt, whnt, conv_bias.reshape(1, EMB),
                     bir, biz, bin_, bhr, bhz, bhn)

    sc_hgather = _make_sc_hgather(n, s_pad)
    hsp = sc_hgather(h, sidxp)

    ops = s2p_w3.shape[0]
    sb_rows = 2000
    stem_preds = pl.pallas_call(
        _tc_stems_body,
        grid=(s // sb_rows,),
        in_specs=[
            pl.BlockSpec((sb_rows, EMB), lambda i: (i, 0)),
            pl.BlockSpec((sb_rows, EMB), lambda i: (i, 0)),
            pl.BlockSpec((EMB, EMB), lambda i: (0, 0)),
            pl.BlockSpec((1, EMB), lambda i: (0, 0)),
            pl.BlockSpec((EMB, EMB), lambda i: (0, 0)),
            pl.BlockSpec((1, EMB), lambda i: (0, 0)),
            pl.BlockSpec((EMB, ops), lambda i: (0, 0)),
            pl.BlockSpec((1, ops), lambda i: (0, 0)),
        ],
        out_specs=pl.BlockSpec((sb_rows, ops), lambda i: (i, 0)),
        out_shape=jax.ShapeDtypeStruct((s, ops), f32),
    )(hsp[:s], st2p[:s], w1at, s2p_b1.reshape(1, EMB), s2p_w2.T,
      s2p_b2.reshape(1, EMB), s2p_w3.T, s2p_b3.reshape(1, ops))

    opm = g2p_w2.shape[0]
    mol_preds = pl.pallas_call(
        _tc_pool_body,
        out_shape=jax.ShapeDtypeStruct((bn, opm), f32),
    )(h.reshape(bn, npg, EMB), g2p_w1.T, g2p_b1.reshape(1, EMB), g2p_w2.T,
      g2p_b2.reshape(1, opm))

    return (mol_preds, stem_preds)


# speedup vs baseline: 3.8891x; 1.2022x over previous
import functools

import jax
import jax.numpy as jnp
from jax import lax
from jax.experimental import pallas as pl
from jax.experimental.pallas import tpu as pltpu
from jax.experimental.pallas import tpu_sc as plsc

EMB = 32
NC = 2
NS = 16
NW = NC * NS
CH = 128
NUM_CONV_STEPS = 4

f32 = jnp.float32
i32 = jnp.int32


def _ceil_to(v, m):
    return (v + m - 1) // m * m


def _lrelu(v):
    return jnp.where(v >= 0, v, 0.01 * v)


def _tc_prep_body(frag_ref, w1t_ref, b1_ref, w2t_ref, b2_ref, stem_ref,
                  w1bt_ref, t1_ref, t2_ref):
    z = _lrelu(jnp.dot(frag_ref[...], w1t_ref[...],
                       preferred_element_type=f32) + b1_ref[...])
    t1_ref[...] = jnp.dot(z, w2t_ref[...], preferred_element_type=f32) + b2_ref[...]
    t2_ref[...] = jnp.dot(stem_ref[...], w1bt_ref[...], preferred_element_type=f32)


def _tc_gru_body(agg_ref, h_ref, degp_ref, wrt_ref,
                 wir_ref, wiz_ref, win_ref, whr_ref, whz_ref, whn_ref,
                 cb_ref, bir_ref, biz_ref, bin_ref, bhr_ref, bhz_ref, bhn_ref,
                 out_ref):
    deg = jnp.sum(degp_ref[...], axis=1)
    deginv = 1.0 / jnp.maximum(deg, 1.0)
    hb = h_ref[...]
    agg = (agg_ref[0] + agg_ref[1]) * deginv[:, None]
    conv = agg + jnp.dot(hb, wrt_ref[...], preferred_element_type=f32) + cb_ref[...]
    m = _lrelu(conv)
    gir = jnp.dot(m, wir_ref[...], preferred_element_type=f32) + bir_ref[...]
    giz = jnp.dot(m, wiz_ref[...], preferred_element_type=f32) + biz_ref[...]
    gin = jnp.dot(m, win_ref[...], preferred_element_type=f32) + bin_ref[...]
    ghr = jnp.dot(hb, whr_ref[...], preferred_element_type=f32) + bhr_ref[...]
    ghz = jnp.dot(hb, whz_ref[...], preferred_element_type=f32) + bhz_ref[...]
    ghn = jnp.dot(hb, whn_ref[...], preferred_element_type=f32) + bhn_ref[...]
    r = jax.nn.sigmoid(gir + ghr)
    z = jax.nn.sigmoid(giz + ghz)
    n = jnp.tanh(gin + r * ghn)
    out_ref[...] = (1.0 - z) * n + z * hb


def _tc_stems_body(hs_ref, st2_ref, w1at_ref, b1_ref, w2t_ref, b2_ref,
                   w3t_ref, b3_ref, out_ref):
    sp = _lrelu(jnp.dot(hs_ref[...], w1at_ref[...], preferred_element_type=f32)
                + st2_ref[...] + b1_ref[...])
    sp = _lrelu(jnp.dot(sp, w2t_ref[...], preferred_element_type=f32) + b2_ref[...])
    out_ref[...] = jnp.dot(sp, w3t_ref[...], preferred_element_type=f32) + b3_ref[...]


def _tc_pool_body(h3_ref, w1t_ref, b1_ref, w2t_ref, b2_ref, out_ref):
    pooled = jnp.mean(h3_ref[...], axis=1)
    mp = _lrelu(jnp.dot(pooled, w1t_ref[...], preferred_element_type=f32) + b1_ref[...])
    out_ref[...] = jnp.dot(mp, w2t_ref[...], preferred_element_type=f32) + b2_ref[...]


def _mesh():
    return plsc.VectorSubcoreMesh(core_axis_name="c", subcore_axis_name="s")


_SC_PARAMS = pltpu.CompilerParams(needs_layout_passes=False,
                                  use_tc_tiling_on_sc=False)


def _wid():
    return lax.axis_index("s") * NC + lax.axis_index("c")


def _make_sc_prep(n_pad, s_pad, n_deg, e_pad):
    nch_n = n_pad // (NW * CH)
    nch_s = s_pad // (NW * CH)
    nch_e = e_pad // (NW * CH)
    et = e_pad // NW

    @functools.partial(
        pl.kernel,
        out_type=(
            jax.ShapeDtypeStruct((n_pad, EMB), f32),
            jax.ShapeDtypeStruct((s_pad, EMB), f32),
            jax.ShapeDtypeStruct((s_pad,), i32),
            jax.ShapeDtypeStruct((NW, (n_deg + 16) // 16, 16), f32),
        ),
        mesh=_mesh(),
        compiler_params=_SC_PARAMS,
        scratch_types=[
            pltpu.VMEM((CH,), i32),
            pltpu.VMEM((CH, EMB), f32),
            pltpu.VMEM((CH,), i32),
            pltpu.VMEM((CH,), i32),
            pltpu.VMEM((16, 16), f32),
            pltpu.VMEM(((n_deg + 16) // 16, 16), f32),
            pltpu.SemaphoreType.DMA,
        ],
    )
    def sc_prep(t1_hbm, t2_hbm, xg_hbm, stg_hbm, sb_hbm, s0_hbm, xsl_hbm,
                dst_hbm, zdeg_hbm,
                h0_hbm, st2_hbm, sidx_hbm, degpart_hbm,
                idxb, rowb, s0b, sidxb, xslv, degv, sem):
        w = _wid()
        ii = lax.iota(i32, 16)
        for k in range(nch_n):
            base = w * (nch_n * CH) + k * CH
            pltpu.sync_copy(xg_hbm.at[pl.ds(base, CH)], idxb)
            pltpu.async_copy(t1_hbm.at[idxb], rowb, sem).wait()
            pltpu.sync_copy(rowb, h0_hbm.at[pl.ds(base, CH)])
        for k in range(nch_s):
            base = w * (nch_s * CH) + k * CH
            pltpu.sync_copy(stg_hbm.at[pl.ds(base, CH)], idxb)
            pltpu.async_copy(t2_hbm.at[idxb], rowb, sem).wait()
            pltpu.sync_copy(rowb, st2_hbm.at[pl.ds(base, CH)])
        pltpu.sync_copy(xsl_hbm, xslv)
        for k in range(nch_s):
            base = w * (nch_s * CH) + k * CH
            pltpu.sync_copy(sb_hbm.at[pl.ds(base, CH)], idxb)
            pltpu.sync_copy(s0_hbm.at[pl.ds(base, CH)], s0b)
            for g in range(CH // 16):
                sb = idxb[pl.ds(g * 16, 16)]
                sv = plsc.bitcast(plsc.load_gather(
                    xslv, [lax.shift_right_logical(sb, 4),
                           lax.bitwise_and(sb, 15)]), i32)
                sidxb[pl.ds(g * 16, 16)] = sv + s0b[pl.ds(g * 16, 16)]
            pltpu.sync_copy(sidxb, sidx_hbm.at[pl.ds(base, CH)])
        pltpu.sync_copy(zdeg_hbm, degv)
        ones16 = jnp.full((16,), 1.0, dtype=f32)
        for k in range(nch_e):
            base = w * et + k * CH
            pltpu.sync_copy(dst_hbm.at[pl.ds(base, CH)], idxb)
            for g in range(CH // 16):
                dvec = idxb[pl.ds(g * 16, 16)]
                plsc.addupdate_scatter(
                    degv, [lax.shift_right_logical(dvec, 4),
                           lax.bitwise_and(dvec, 15)], ones16)
        pltpu.sync_copy(degv, degpart_hbm.at[w])

    return sc_prep


def _make_sc_edge(n, n_agg, e_pad, n_bond):
    et = e_pad // NW
    nch_e = et // CH
    zrows = n_agg // NS
    orows = n // NS

    npair = nch_e // 2

    @functools.partial(
        pl.kernel,
        out_type=jax.ShapeDtypeStruct((NC, n, EMB), f32),
        mesh=_mesh(),
        compiler_params=_SC_PARAMS,
        scratch_types=[
            pltpu.VMEM((4, CH), i32),
            pltpu.VMEM((4, CH), i32),
            pltpu.VMEM((CH,), i32),
            pltpu.VMEM((CH,), i32),
            pltpu.VMEM((CH, EMB), f32),
            pltpu.VMEM((CH, EMB), f32),
            pltpu.VMEM((CH, EMB), f32),
            pltpu.VMEM((CH, EMB), f32),
            pltpu.VMEM((n_bond, EMB), f32),
            pltpu.VMEM_SHARED((n_agg, EMB), f32),
            pltpu.SemaphoreType.DMA,
            pltpu.SemaphoreType.DMA,
            pltpu.SemaphoreType.DMA,
            pltpu.SemaphoreType.DMA,
            pltpu.SemaphoreType.DMA,
            pltpu.SemaphoreType.DMA,
        ],
    )
    def sc_edge(h_hbm, idx4_hbm, bond_hbm, z_hbm,
                agg_hbm,
                islot0, islot1, dstb0, dstb1, gbuf0, gbuf1, msg0, msg1,
                tabv, aggs, semi0, semi1, semg0, semg1, sems0, sems1):
        cid = lax.axis_index("c")
        sid = lax.axis_index("s")
        w = sid * NC + cid
        ii = lax.iota(i32, 16)
        islot = (islot0, islot1)
        dstb = (dstb0, dstb1)
        gbuf = (gbuf0, gbuf1)
        msg = (msg0, msg1)
        semi = (semi0, semi1)
        semg = (semg0, semg1)
        sems = (sems0, sems1)
        ct0 = w * nch_e
        pltpu.sync_copy(z_hbm.at[pl.ds(sid * zrows, zrows)],
                        aggs.at[pl.ds(sid * zrows, zrows)])
        pltpu.sync_copy(bond_hbm, tabv)
        plsc.subcore_barrier()

        def compute(b):
            for g in range(CH // 16):
                gi = ii + (g * 16)
                ev0 = islot[b][2, pl.ds(g * 16, 16)]
                ev1 = islot[b][3, pl.ds(g * 16, 16)]
                acc = [jnp.zeros((16,), f32) for _ in range(4)]
                for c in range(EMB):
                    cv = jnp.full((16,), c, dtype=i32)
                    gc = plsc.load_gather(gbuf[b], [gi, cv])
                    ac = plsc.load_gather(tabv, [ev0, cv])
                    acc[c & 3] = acc[c & 3] + gc * ac
                s = (acc[0] + acc[1]) + (acc[2] + acc[3])
                for c in range(EMB):
                    cv = jnp.full((16,), c, dtype=i32)
                    bc = plsc.load_gather(tabv, [ev1, cv])
                    plsc.store_scatter(msg[b], [gi, cv], s * bc)

        pltpu.sync_copy(idx4_hbm.at[ct0], islot0)
        pltpu.async_copy(idx4_hbm.at[ct0 + 1], islot1, semi1)
        pltpu.async_copy(h_hbm.at[islot0.at[0]], gbuf0, semg0)

        def pair(i, carry):
            for b in (0, 1):
                c = 2 * i + b
                nb = 1 - b
                @pl.when(i >= 1)
                def _():
                    pltpu.make_async_copy(
                        msg[b], aggs.at[pl.ds(0, CH)], sems[b]).wait()
                pltpu.make_async_copy(
                    h_hbm.at[pl.ds(0, CH)], gbuf[b], semg[b]).wait()
                for g in range(CH // 16):
                    dstb[b][pl.ds(g * 16, 16)] = islot[b][1, pl.ds(g * 16, 16)]
                @pl.when(c + 1 < nch_e)
                def _():
                    pltpu.make_async_copy(
                        idx4_hbm.at[ct0], islot[nb], semi[nb]).wait()
                    pltpu.async_copy(
                        h_hbm.at[islot[nb].at[0]], gbuf[nb], semg[nb])
                compute(b)
                pltpu.async_copy(msg[b], aggs.at[dstb[b]], sems[b], add=True)
                @pl.when(c + 2 < nch_e)
                def _():
                    pltpu.async_copy(
                        idx4_hbm.at[ct0 + c + 2], islot[b], semi[b])
            return carry

        lax.fori_loop(0, npair, pair, 0)
        pltpu.make_async_copy(msg0, aggs.at[pl.ds(0, CH)], sems0).wait()
        pltpu.make_async_copy(msg1, aggs.at[pl.ds(0, CH)], sems1).wait()
        plsc.subcore_barrier()
        pltpu.sync_copy(aggs.at[pl.ds(sid * orows, orows)],
                        agg_hbm.at[cid, pl.ds(sid * orows, orows)])

    return sc_edge


def _make_sc_hgather(n, s_pad):
    nch_s = s_pad // (NW * CH)

    @functools.partial(
        pl.kernel,
        out_type=jax.ShapeDtypeStruct((s_pad, EMB), f32),
        mesh=_mesh(),
        compiler_params=_SC_PARAMS,
        scratch_types=[
            pltpu.VMEM((CH,), i32),
            pltpu.VMEM((CH, EMB), f32),
            pltpu.SemaphoreType.DMA,
        ],
    )
    def sc_hgather(h_hbm, sidx_hbm, out_hbm, idxb, rowb, sem):
        w = _wid()
        for k in range(nch_s):
            base = w * (nch_s * CH) + k * CH
            pltpu.sync_copy(sidx_hbm.at[pl.ds(base, CH)], idxb)
            pltpu.async_copy(h_hbm.at[idxb], rowb, sem).wait()
            pltpu.sync_copy(rowb, out_hbm.at[pl.ds(base, CH)])

    return sc_hgather


def kernel(x, edge_index, edge_attrs, stem_types, stems, stems_batch, batch,
           x_slices, frag_emb_w, stem_emb_w, bond_emb_w, conv_root_w,
           conv_bias, gru_w_ih, gru_w_hh, gru_b_ih, gru_b_hh, f2e_w1, f2e_b1,
           f2e_w2, f2e_b2, s2p_w1, s2p_b1, s2p_w2, s2p_b2, s2p_w3, s2p_b3,
           g2p_w1, g2p_b1, g2p_w2, g2p_b2):
    n = x.shape[0]
    e = edge_index.shape[1]
    s = stem_types.shape[0]
    bn = x_slices.shape[0] - 1
    n_bond = bond_emb_w.shape[0]
    npg = n // bn

    e_pad = _ceil_to(e, NW * CH)
    n_pad = _ceil_to(n, NW * CH)
    s_pad = _ceil_to(s, NW * CH)
    n_agg = _ceil_to(n + 16, NS)

    srcp = jnp.concatenate([edge_index[0], jnp.zeros((e_pad - e,), i32)])
    dstp = jnp.concatenate([edge_index[1], jnp.full((e_pad - e,), n, i32)])
    e0p = jnp.concatenate([edge_attrs[:, 0], jnp.zeros((e_pad - e,), i32)])
    e1p = jnp.concatenate([edge_attrs[:, 1], jnp.zeros((e_pad - e,), i32)])
    xgp = jnp.concatenate([x, jnp.zeros((n_pad - n,), i32)])
    stgp = jnp.concatenate([stem_types, jnp.zeros((s_pad - s,), i32)])
    sbp = jnp.concatenate([stems_batch, jnp.zeros((s_pad - s,), i32)])
    s0p = jnp.concatenate([stems[:, 0], jnp.zeros((s_pad - s,), i32)])
    xslp = jnp.concatenate(
        [x_slices, jnp.zeros((256 - x_slices.shape[0],), i32)]).reshape(16, 16)
    xslp = lax.bitcast_convert_type(xslp, f32)
    zeros_agg = jnp.zeros((n_agg, EMB), f32)
    zeros_deg = jnp.zeros(((n + 16) // 16, 16), f32)

    w1at = s2p_w1[:, :EMB].T
    w1bt = s2p_w1[:, EMB:].T
    wirt, wizt, wint = (gru_w_ih[0:EMB].T, gru_w_ih[EMB:2 * EMB].T,
                        gru_w_ih[2 * EMB:].T)
    whrt, whzt, whnt = (gru_w_hh[0:EMB].T, gru_w_hh[EMB:2 * EMB].T,
                        gru_w_hh[2 * EMB:].T)
    bir, biz, bin_ = (gru_b_ih[0:EMB].reshape(1, EMB),
                      gru_b_ih[EMB:2 * EMB].reshape(1, EMB),
                      gru_b_ih[2 * EMB:].reshape(1, EMB))
    bhr, bhz, bhn = (gru_b_hh[0:EMB].reshape(1, EMB),
                     gru_b_hh[EMB:2 * EMB].reshape(1, EMB),
                     gru_b_hh[2 * EMB:].reshape(1, EMB))

    nf = frag_emb_w.shape[0]
    nst = stem_emb_w.shape[0]
    t1, t2 = pl.pallas_call(
        _tc_prep_body,
        out_shape=(jax.ShapeDtypeStruct((nf, EMB), f32),
                   jax.ShapeDtypeStruct((nst, EMB), f32)),
    )(frag_emb_w, f2e_w1.T, f2e_b1.reshape(1, EMB), f2e_w2.T,
      f2e_b2.reshape(1, EMB), stem_emb_w, w1bt)

    sc_prep = _make_sc_prep(n_pad, s_pad, n, e_pad)
    h0p, st2p, sidxp, degpart = sc_prep(
        t1, t2, xgp, stgp, sbp, s0p, xslp, dstp, zeros_deg)
    h = h0p[:n]

    sc_edge = _make_sc_edge(n, n_agg, e_pad, n_bond)
    rb = 1000
    grid = n // rb
    gru_call = pl.pallas_call(
        _tc_gru_body,
        grid=(grid,),
        in_specs=[
            pl.BlockSpec((NC, rb, EMB), lambda i: (0, i, 0)),
            pl.BlockSpec((rb, EMB), lambda i: (i, 0)),
            pl.BlockSpec((rb, NW), lambda i: (i, 0)),
        ] + [pl.BlockSpec((EMB, EMB), lambda i: (0, 0))] * 7
          + [pl.BlockSpec((1, EMB), lambda i: (0, 0))] * 7,
        out_specs=pl.BlockSpec((rb, EMB), lambda i: (i, 0)),
        out_shape=jax.ShapeDtypeStruct((n, EMB), f32),
    )
    degpt = degpart.reshape(NW, -1)[:, :n].T
    idx4 = jnp.stack([srcp, dstp, e0p, e1p]).reshape(
        4, e_pad // CH, CH).transpose(1, 0, 2)
    for _ in range(NUM_CONV_STEPS):
        aggp = sc_edge(h, idx4, bond_emb_w, zeros_agg)
        h = gru_call(aggp, h, degpt, conv_root_w.T, wirt, wizt, wint,
                     whrt, whzt, whnt, conv_bias.reshape(1, EMB),
                     bir, biz, bin_, bhr, bhz, bhn)

    sc_hgather = _make_sc_hgather(n, s_pad)
    hsp = sc_hgather(h, sidxp)

    ops = s2p_w3.shape[0]
    sb_rows = 2000
    stem_preds = pl.pallas_call(
        _tc_stems_body,
        grid=(s // sb_rows,),
        in_specs=[
            pl.BlockSpec((sb_rows, EMB), lambda i: (i, 0)),
            pl.BlockSpec((sb_rows, EMB), lambda i: (i, 0)),
            pl.BlockSpec((EMB, EMB), lambda i: (0, 0)),
            pl.BlockSpec((1, EMB), lambda i: (0, 0)),
            pl.BlockSpec((EMB, EMB), lambda i: (0, 0)),
            pl.BlockSpec((1, EMB), lambda i: (0, 0)),
            pl.BlockSpec((EMB, ops), lambda i: (0, 0)),
            pl.BlockSpec((1, ops), lambda i: (0, 0)),
        ],
        out_specs=pl.BlockSpec((sb_rows, ops), lambda i: (i, 0)),
        out_shape=jax.ShapeDtypeStruct((s, ops), f32),
    )(hsp[:s], st2p[:s], w1at, s2p_b1.reshape(1, EMB), s2p_w2.T,
      s2p_b2.reshape(1, EMB), s2p_w3.T, s2p_b3.reshape(1, ops))

    opm = g2p_w2.shape[0]
    mol_preds = pl.pallas_call(
        _tc_pool_body,
        out_shape=jax.ShapeDtypeStruct((bn, opm), f32),
    )(h.reshape(bn, npg, EMB), g2p_w1.T, g2p_b1.reshape(1, EMB), g2p_w2.T,
      g2p_b2.reshape(1, opm))

    return (mol_preds, stem_preds)


# speedup vs baseline: 6.0379x; 1.5525x over previous
import functools

import jax
import jax.numpy as jnp
from jax import lax
from jax.experimental import pallas as pl
from jax.experimental.pallas import tpu as pltpu
from jax.experimental.pallas import tpu_sc as plsc

EMB = 32
NC = 2
NS = 16
NW = NC * NS
CH = 128
NUM_CONV_STEPS = 4

f32 = jnp.float32
i32 = jnp.int32


def _ceil_to(v, m):
    return (v + m - 1) // m * m


def _lrelu(v):
    return jnp.where(v >= 0, v, 0.01 * v)


def _tc_prep_body(frag_ref, w1t_ref, b1_ref, w2t_ref, b2_ref, stem_ref,
                  w1bt_ref, t1_ref, t2_ref):
    z = _lrelu(jnp.dot(frag_ref[...], w1t_ref[...],
                       preferred_element_type=f32) + b1_ref[...])
    t1_ref[...] = jnp.dot(z, w2t_ref[...], preferred_element_type=f32) + b2_ref[...]
    t2_ref[...] = jnp.dot(stem_ref[...], w1bt_ref[...], preferred_element_type=f32)


def _tc_gru_body(agg_ref, h_ref, degp_ref, wrt_ref,
                 wir_ref, wiz_ref, win_ref, whr_ref, whz_ref, whn_ref,
                 cb_ref, bir_ref, biz_ref, bin_ref, bhr_ref, bhz_ref, bhn_ref,
                 out_ref):
    deg = jnp.sum(degp_ref[...], axis=1)
    deginv = 1.0 / jnp.maximum(deg, 1.0)
    hb = h_ref[...]
    agg = (agg_ref[0] + agg_ref[1]) * deginv[:, None]
    conv = agg + jnp.dot(hb, wrt_ref[...], preferred_element_type=f32) + cb_ref[...]
    m = _lrelu(conv)
    gir = jnp.dot(m, wir_ref[...], preferred_element_type=f32) + bir_ref[...]
    giz = jnp.dot(m, wiz_ref[...], preferred_element_type=f32) + biz_ref[...]
    gin = jnp.dot(m, win_ref[...], preferred_element_type=f32) + bin_ref[...]
    ghr = jnp.dot(hb, whr_ref[...], preferred_element_type=f32) + bhr_ref[...]
    ghz = jnp.dot(hb, whz_ref[...], preferred_element_type=f32) + bhz_ref[...]
    ghn = jnp.dot(hb, whn_ref[...], preferred_element_type=f32) + bhn_ref[...]
    r = jax.nn.sigmoid(gir + ghr)
    z = jax.nn.sigmoid(giz + ghz)
    n = jnp.tanh(gin + r * ghn)
    out_ref[...] = (1.0 - z) * n + z * hb


def _tc_stems_body(hs_ref, st2_ref, w1at_ref, b1_ref, w2t_ref, b2_ref,
                   w3t_ref, b3_ref, out_ref):
    sp = _lrelu(jnp.dot(hs_ref[...], w1at_ref[...], preferred_element_type=f32)
                + st2_ref[...] + b1_ref[...])
    sp = _lrelu(jnp.dot(sp, w2t_ref[...], preferred_element_type=f32) + b2_ref[...])
    out_ref[...] = jnp.dot(sp, w3t_ref[...], preferred_element_type=f32) + b3_ref[...]


def _tc_pool_body(h3_ref, w1t_ref, b1_ref, w2t_ref, b2_ref, out_ref):
    pooled = jnp.mean(h3_ref[...], axis=1)
    mp = _lrelu(jnp.dot(pooled, w1t_ref[...], preferred_element_type=f32) + b1_ref[...])
    out_ref[...] = jnp.dot(mp, w2t_ref[...], preferred_element_type=f32) + b2_ref[...]


def _mesh():
    return plsc.VectorSubcoreMesh(core_axis_name="c", subcore_axis_name="s")


_SC_PARAMS = pltpu.CompilerParams(needs_layout_passes=False,
                                  use_tc_tiling_on_sc=False)


def _wid():
    return lax.axis_index("s") * NC + lax.axis_index("c")


def _make_sc_prep(n_pad, s_pad, n_deg, e_pad):
    nch_n = n_pad // (NW * CH)
    nch_s = s_pad // (NW * CH)
    nch_e = e_pad // (NW * CH)
    et = e_pad // NW

    @functools.partial(
        pl.kernel,
        out_type=(
            jax.ShapeDtypeStruct((n_pad, EMB), f32),
            jax.ShapeDtypeStruct((s_pad, EMB), f32),
            jax.ShapeDtypeStruct((s_pad,), i32),
            jax.ShapeDtypeStruct((NW, (n_deg + 16) // 16, 16), f32),
        ),
        mesh=_mesh(),
        compiler_params=_SC_PARAMS,
        scratch_types=[
            pltpu.VMEM((CH,), i32),
            pltpu.VMEM((CH, EMB), f32),
            pltpu.VMEM((CH,), i32),
            pltpu.VMEM((CH,), i32),
            pltpu.VMEM((16, 16), f32),
            pltpu.VMEM(((n_deg + 16) // 16, 16), f32),
            pltpu.SemaphoreType.DMA,
        ],
    )
    def sc_prep(t1_hbm, t2_hbm, xg_hbm, stg_hbm, sb_hbm, s0_hbm, xsl_hbm,
                dst_hbm, zdeg_hbm,
                h0_hbm, st2_hbm, sidx_hbm, degpart_hbm,
                idxb, rowb, s0b, sidxb, xslv, degv, sem):
        w = _wid()
        ii = lax.iota(i32, 16)
        for k in range(nch_n):
            base = w * (nch_n * CH) + k * CH
            pltpu.sync_copy(xg_hbm.at[pl.ds(base, CH)], idxb)
            pltpu.async_copy(t1_hbm.at[idxb], rowb, sem).wait()
            pltpu.sync_copy(rowb, h0_hbm.at[pl.ds(base, CH)])
        for k in range(nch_s):
            base = w * (nch_s * CH) + k * CH
            pltpu.sync_copy(stg_hbm.at[pl.ds(base, CH)], idxb)
            pltpu.async_copy(t2_hbm.at[idxb], rowb, sem).wait()
            pltpu.sync_copy(rowb, st2_hbm.at[pl.ds(base, CH)])
        pltpu.sync_copy(xsl_hbm, xslv)
        for k in range(nch_s):
            base = w * (nch_s * CH) + k * CH
            pltpu.sync_copy(sb_hbm.at[pl.ds(base, CH)], idxb)
            pltpu.sync_copy(s0_hbm.at[pl.ds(base, CH)], s0b)
            for g in range(CH // 16):
                sb = idxb[pl.ds(g * 16, 16)]
                sv = plsc.bitcast(plsc.load_gather(
                    xslv, [lax.shift_right_logical(sb, 4),
                           lax.bitwise_and(sb, 15)]), i32)
                sidxb[pl.ds(g * 16, 16)] = sv + s0b[pl.ds(g * 16, 16)]
            pltpu.sync_copy(sidxb, sidx_hbm.at[pl.ds(base, CH)])
        pltpu.sync_copy(zdeg_hbm, degv)
        ones16 = jnp.full((16,), 1.0, dtype=f32)
        for k in range(nch_e):
            base = w * et + k * CH
            pltpu.sync_copy(dst_hbm.at[pl.ds(base, CH)], idxb)
            for g in range(CH // 16):
                dvec = idxb[pl.ds(g * 16, 16)]
                plsc.addupdate_scatter(
                    degv, [lax.shift_right_logical(dvec, 4),
                           lax.bitwise_and(dvec, 15)], ones16)
        pltpu.sync_copy(degv, degpart_hbm.at[w])

    return sc_prep


def _make_sc_edge(n, n_agg, e_pad, n_bond):
    et = e_pad // NW
    nch_e = et // CH
    zrows = n_agg // NS
    orows = n // NS

    npair = nch_e // 2

    @functools.partial(
        pl.kernel,
        out_type=jax.ShapeDtypeStruct((NC, n, EMB), f32),
        mesh=_mesh(),
        compiler_params=_SC_PARAMS,
        scratch_types=[
            pltpu.VMEM((4, CH), i32),
            pltpu.VMEM((4, CH), i32),
            pltpu.VMEM((CH,), i32),
            pltpu.VMEM((CH,), i32),
            pltpu.VMEM((CH, EMB), f32),
            pltpu.VMEM((CH, EMB), f32),
            pltpu.VMEM((CH, EMB), f32),
            pltpu.VMEM((CH, EMB), f32),
            pltpu.VMEM((CH, EMB), f32),
            pltpu.VMEM((CH, EMB), f32),
            pltpu.VMEM((CH, EMB), f32),
            pltpu.VMEM((CH, EMB), f32),
            pltpu.VMEM_SHARED((n_agg, EMB), f32),
            pltpu.SemaphoreType.DMA,
            pltpu.SemaphoreType.DMA,
            pltpu.SemaphoreType.DMA,
            pltpu.SemaphoreType.DMA,
            pltpu.SemaphoreType.DMA,
            pltpu.SemaphoreType.DMA,
        ],
    )
    def sc_edge(h_hbm, idx4_hbm, bond_hbm, z_hbm,
                agg_hbm,
                islot0, islot1, dstb0, dstb1, gbuf0, gbuf1, msg0, msg1,
                abuf0, abuf1, bbuf0, bbuf1,
                aggs, semi0, semi1, semg0, semg1, sems0, sems1):
        cid = lax.axis_index("c")
        sid = lax.axis_index("s")
        w = sid * NC + cid
        islot = (islot0, islot1)
        dstb = (dstb0, dstb1)
        gbuf = (gbuf0, gbuf1)
        msg = (msg0, msg1)
        abuf = (abuf0, abuf1)
        bbuf = (bbuf0, bbuf1)
        semi = (semi0, semi1)
        semg = (semg0, semg1)
        sems = (sems0, sems1)
        ct0 = w * nch_e
        pltpu.sync_copy(z_hbm.at[pl.ds(sid * zrows, zrows)],
                        aggs.at[pl.ds(sid * zrows, zrows)])
        plsc.subcore_barrier()

        def issue_gathers(b):
            pltpu.async_copy(h_hbm.at[islot[b].at[0]], gbuf[b], semg[b])
            pltpu.async_copy(bond_hbm.at[islot[b].at[2]], abuf[b], semg[b])
            pltpu.async_copy(bond_hbm.at[islot[b].at[3]], bbuf[b], semg[b])

        def wait_gathers(b):
            for _ in range(3):
                pltpu.make_async_copy(
                    h_hbm.at[pl.ds(0, CH)], gbuf[b], semg[b]).wait()

        def compute(b):
            for e in range(CH):
                g0 = gbuf[b][e, pl.ds(0, 16)]
                g1 = gbuf[b][e, pl.ds(16, 16)]
                a0 = abuf[b][e, pl.ds(0, 16)]
                a1 = abuf[b][e, pl.ds(16, 16)]
                s = jnp.sum(g0 * a0 + g1 * a1)
                msg[b][e, pl.ds(0, 16)] = s * bbuf[b][e, pl.ds(0, 16)]
                msg[b][e, pl.ds(16, 16)] = s * bbuf[b][e, pl.ds(16, 16)]

        pltpu.sync_copy(idx4_hbm.at[ct0], islot0)
        pltpu.async_copy(idx4_hbm.at[ct0 + 1], islot1, semi1)
        issue_gathers(0)

        def pair(i, carry):
            for b in (0, 1):
                c = 2 * i + b
                nb = 1 - b
                @pl.when(i >= 1)
                def _():
                    pltpu.make_async_copy(
                        msg[b], aggs.at[pl.ds(0, CH)], sems[b]).wait()
                wait_gathers(b)
                for g in range(CH // 16):
                    dstb[b][pl.ds(g * 16, 16)] = islot[b][1, pl.ds(g * 16, 16)]
                @pl.when(c + 1 < nch_e)
                def _():
                    pltpu.make_async_copy(
                        idx4_hbm.at[ct0], islot[nb], semi[nb]).wait()
                    issue_gathers(nb)
                compute(b)
                pltpu.async_copy(msg[b], aggs.at[dstb[b]], sems[b], add=True)
                @pl.when(c + 2 < nch_e)
                def _():
                    pltpu.async_copy(
                        idx4_hbm.at[ct0 + c + 2], islot[b], semi[b])
            return carry

        lax.fori_loop(0, npair, pair, 0)
        pltpu.make_async_copy(msg0, aggs.at[pl.ds(0, CH)], sems0).wait()
        pltpu.make_async_copy(msg1, aggs.at[pl.ds(0, CH)], sems1).wait()
        plsc.subcore_barrier()
        pltpu.sync_copy(aggs.at[pl.ds(sid * orows, orows)],
                        agg_hbm.at[cid, pl.ds(sid * orows, orows)])

    return sc_edge


def _make_sc_hgather(n, s_pad):
    nch_s = s_pad // (NW * CH)

    @functools.partial(
        pl.kernel,
        out_type=jax.ShapeDtypeStruct((s_pad, EMB), f32),
        mesh=_mesh(),
        compiler_params=_SC_PARAMS,
        scratch_types=[
            pltpu.VMEM((CH,), i32),
            pltpu.VMEM((CH, EMB), f32),
            pltpu.SemaphoreType.DMA,
        ],
    )
    def sc_hgather(h_hbm, sidx_hbm, out_hbm, idxb, rowb, sem):
        w = _wid()
        for k in range(nch_s):
            base = w * (nch_s * CH) + k * CH
            pltpu.sync_copy(sidx_hbm.at[pl.ds(base, CH)], idxb)
            pltpu.async_copy(h_hbm.at[idxb], rowb, sem).wait()
            pltpu.sync_copy(rowb, out_hbm.at[pl.ds(base, CH)])

    return sc_hgather


def kernel(x, edge_index, edge_attrs, stem_types, stems, stems_batch, batch,
           x_slices, frag_emb_w, stem_emb_w, bond_emb_w, conv_root_w,
           conv_bias, gru_w_ih, gru_w_hh, gru_b_ih, gru_b_hh, f2e_w1, f2e_b1,
           f2e_w2, f2e_b2, s2p_w1, s2p_b1, s2p_w2, s2p_b2, s2p_w3, s2p_b3,
           g2p_w1, g2p_b1, g2p_w2, g2p_b2):
    n = x.shape[0]
    e = edge_index.shape[1]
    s = stem_types.shape[0]
    bn = x_slices.shape[0] - 1
    n_bond = bond_emb_w.shape[0]
    npg = n // bn

    e_pad = _ceil_to(e, NW * CH)
    n_pad = _ceil_to(n, NW * CH)
    s_pad = _ceil_to(s, NW * CH)
    n_agg = _ceil_to(n + 16, NS)

    srcp = jnp.concatenate([edge_index[0], jnp.zeros((e_pad - e,), i32)])
    dstp = jnp.concatenate([edge_index[1], jnp.full((e_pad - e,), n, i32)])
    e0p = jnp.concatenate([edge_attrs[:, 0], jnp.zeros((e_pad - e,), i32)])
    e1p = jnp.concatenate([edge_attrs[:, 1], jnp.zeros((e_pad - e,), i32)])
    xgp = jnp.concatenate([x, jnp.zeros((n_pad - n,), i32)])
    stgp = jnp.concatenate([stem_types, jnp.zeros((s_pad - s,), i32)])
    sbp = jnp.concatenate([stems_batch, jnp.zeros((s_pad - s,), i32)])
    s0p = jnp.concatenate([stems[:, 0], jnp.zeros((s_pad - s,), i32)])
    xslp = jnp.concatenate(
        [x_slices, jnp.zeros((256 - x_slices.shape[0],), i32)]).reshape(16, 16)
    xslp = lax.bitcast_convert_type(xslp, f32)
    zeros_agg = jnp.zeros((n_agg, EMB), f32)
    zeros_deg = jnp.zeros(((n + 16) // 16, 16), f32)

    w1at = s2p_w1[:, :EMB].T
    w1bt = s2p_w1[:, EMB:].T
    wirt, wizt, wint = (gru_w_ih[0:EMB].T, gru_w_ih[EMB:2 * EMB].T,
                        gru_w_ih[2 * EMB:].T)
    whrt, whzt, whnt = (gru_w_hh[0:EMB].T, gru_w_hh[EMB:2 * EMB].T,
                        gru_w_hh[2 * EMB:].T)
    bir, biz, bin_ = (gru_b_ih[0:EMB].reshape(1, EMB),
                      gru_b_ih[EMB:2 * EMB].reshape(1, EMB),
                      gru_b_ih[2 * EMB:].reshape(1, EMB))
    bhr, bhz, bhn = (gru_b_hh[0:EMB].reshape(1, EMB),
                     gru_b_hh[EMB:2 * EMB].reshape(1, EMB),
                     gru_b_hh[2 * EMB:].reshape(1, EMB))

    nf = frag_emb_w.shape[0]
    nst = stem_emb_w.shape[0]
    t1, t2 = pl.pallas_call(
        _tc_prep_body,
        out_shape=(jax.ShapeDtypeStruct((nf, EMB), f32),
                   jax.ShapeDtypeStruct((nst, EMB), f32)),
    )(frag_emb_w, f2e_w1.T, f2e_b1.reshape(1, EMB), f2e_w2.T,
      f2e_b2.reshape(1, EMB), stem_emb_w, w1bt)

    sc_prep = _make_sc_prep(n_pad, s_pad, n, e_pad)
    h0p, st2p, sidxp, degpart = sc_prep(
        t1, t2, xgp, stgp, sbp, s0p, xslp, dstp, zeros_deg)
    h = h0p[:n]

    sc_edge = _make_sc_edge(n, n_agg, e_pad, n_bond)
    rb = 1000
    grid = n // rb
    gru_call = pl.pallas_call(
        _tc_gru_body,
        grid=(grid,),
        in_specs=[
            pl.BlockSpec((NC, rb, EMB), lambda i: (0, i, 0)),
            pl.BlockSpec((rb, EMB), lambda i: (i, 0)),
            pl.BlockSpec((rb, NW), lambda i: (i, 0)),
        ] + [pl.BlockSpec((EMB, EMB), lambda i: (0, 0))] * 7
          + [pl.BlockSpec((1, EMB), lambda i: (0, 0))] * 7,
        out_specs=pl.BlockSpec((rb, EMB), lambda i: (i, 0)),
        out_shape=jax.ShapeDtypeStruct((n, EMB), f32),
    )
    degpt = degpart.reshape(NW, -1)[:, :n].T
    idx4 = jnp.stack([srcp, dstp, e0p, e1p]).reshape(
        4, e_pad // CH, CH).transpose(1, 0, 2)
    for _ in range(NUM_CONV_STEPS):
        aggp = sc_edge(h, idx4, bond_emb_w, zeros_agg)
        h = gru_call(aggp, h, degpt, conv_root_w.T, wirt, wizt, wint,
                     whrt, whzt, whnt, conv_bias.reshape(1, EMB),
                     bir, biz, bin_, bhr, bhz, bhn)

    sc_hgather = _make_sc_hgather(n, s_pad)
    hsp = sc_hgather(h, sidxp)

    ops = s2p_w3.shape[0]
    sb_rows = 2000
    stem_preds = pl.pallas_call(
        _tc_stems_body,
        grid=(s // sb_rows,),
        in_specs=[
            pl.BlockSpec((sb_rows, EMB), lambda i: (i, 0)),
            pl.BlockSpec((sb_rows, EMB), lambda i: (i, 0)),
            pl.BlockSpec((EMB, EMB), lambda i: (0, 0)),
            pl.BlockSpec((1, EMB), lambda i: (0, 0)),
            pl.BlockSpec((EMB, EMB), lambda i: (0, 0)),
            pl.BlockSpec((1, EMB), lambda i: (0, 0)),
            pl.BlockSpec((EMB, ops), lambda i: (0, 0)),
            pl.BlockSpec((1, ops), lambda i: (0, 0)),
        ],
        out_specs=pl.BlockSpec((sb_rows, ops), lambda i: (i, 0)),
        out_shape=jax.ShapeDtypeStruct((s, ops), f32),
    )(hsp[:s], st2p[:s], w1at, s2p_b1.reshape(1, EMB), s2p_w2.T,
      s2p_b2.reshape(1, EMB), s2p_w3.T, s2p_b3.reshape(1, ops))

    opm = g2p_w2.shape[0]
    mol_preds = pl.pallas_call(
        _tc_pool_body,
        out_shape=jax.ShapeDtypeStruct((bn, opm), f32),
    )(h.reshape(bn, npg, EMB), g2p_w1.T, g2p_b1.reshape(1, EMB), g2p_w2.T,
      g2p_b2.reshape(1, opm))

    return (mol_preds, stem_preds)


# speedup vs baseline: 7.9379x; 1.3147x over previous
import functools

import jax
import jax.numpy as jnp
from jax import lax
from jax.experimental import pallas as pl
from jax.experimental.pallas import tpu as pltpu
from jax.experimental.pallas import tpu_sc as plsc

EMB = 32
NC = 2
NS = 16
NW = NC * NS
CH = 128
NUM_CONV_STEPS = 4

f32 = jnp.float32
i32 = jnp.int32


def _ceil_to(v, m):
    return (v + m - 1) // m * m


def _lrelu(v):
    return jnp.where(v >= 0, v, 0.01 * v)


def _tc_prep_body(frag_ref, w1t_ref, b1_ref, w2t_ref, b2_ref, stem_ref,
                  w1bt_ref, t1_ref, t2_ref):
    z = _lrelu(jnp.dot(frag_ref[...], w1t_ref[...],
                       preferred_element_type=f32) + b1_ref[...])
    t1_ref[...] = jnp.dot(z, w2t_ref[...], preferred_element_type=f32) + b2_ref[...]
    t2_ref[...] = jnp.dot(stem_ref[...], w1bt_ref[...], preferred_element_type=f32)


def _tc_gru_body(aggz_ref, h_ref, degp_ref, bond_ref, bondt_ref, wrt_ref,
                 wir_ref, wiz_ref, win_ref, whr_ref, whz_ref, whn_ref,
                 cb_ref, bir_ref, biz_ref, bin_ref, bhr_ref, bhz_ref, bhn_ref,
                 out_ref, p_ref):
    deg = jnp.sum(degp_ref[...], axis=1)
    deginv = 1.0 / jnp.maximum(deg, 1.0)
    hb = h_ref[...]
    zsum = aggz_ref[0] + aggz_ref[1]
    agg = jnp.dot(zsum, bond_ref[...],
                  preferred_element_type=f32) * deginv[:, None]
    conv = agg + jnp.dot(hb, wrt_ref[...], preferred_element_type=f32) + cb_ref[...]
    m = _lrelu(conv)
    gir = jnp.dot(m, wir_ref[...], preferred_element_type=f32) + bir_ref[...]
    giz = jnp.dot(m, wiz_ref[...], preferred_element_type=f32) + biz_ref[...]
    gin = jnp.dot(m, win_ref[...], preferred_element_type=f32) + bin_ref[...]
    ghr = jnp.dot(hb, whr_ref[...], preferred_element_type=f32) + bhr_ref[...]
    ghz = jnp.dot(hb, whz_ref[...], preferred_element_type=f32) + bhz_ref[...]
    ghn = jnp.dot(hb, whn_ref[...], preferred_element_type=f32) + bhn_ref[...]
    r = jax.nn.sigmoid(gir + ghr)
    z = jax.nn.sigmoid(giz + ghz)
    n = jnp.tanh(gin + r * ghn)
    hn = (1.0 - z) * n + z * hb
    out_ref[...] = hn
    p_ref[...] = jnp.dot(hn, bondt_ref[...], preferred_element_type=f32)


def _tc_p0_body(h_ref, bondt_ref, out_ref):
    out_ref[...] = jnp.dot(h_ref[...], bondt_ref[...],
                           preferred_element_type=f32)


def _tc_stems_body(hs_ref, st2_ref, w1at_ref, b1_ref, w2t_ref, b2_ref,
                   w3t_ref, b3_ref, out_ref):
    sp = _lrelu(jnp.dot(hs_ref[...], w1at_ref[...], preferred_element_type=f32)
                + st2_ref[...] + b1_ref[...])
    sp = _lrelu(jnp.dot(sp, w2t_ref[...], preferred_element_type=f32) + b2_ref[...])
    out_ref[...] = jnp.dot(sp, w3t_ref[...], preferred_element_type=f32) + b3_ref[...]


def _tc_pool_body(h3_ref, w1t_ref, b1_ref, w2t_ref, b2_ref, out_ref):
    pooled = jnp.mean(h3_ref[...], axis=1)
    mp = _lrelu(jnp.dot(pooled, w1t_ref[...], preferred_element_type=f32) + b1_ref[...])
    out_ref[...] = jnp.dot(mp, w2t_ref[...], preferred_element_type=f32) + b2_ref[...]


def _mesh():
    return plsc.VectorSubcoreMesh(core_axis_name="c", subcore_axis_name="s")


_SC_PARAMS = pltpu.CompilerParams(needs_layout_passes=False,
                                  use_tc_tiling_on_sc=False)


def _wid():
    return lax.axis_index("s") * NC + lax.axis_index("c")


def _make_sc_prep(n_pad, s_pad, n_deg, e_pad, n_bond):
    nch_n = n_pad // (NW * CH)
    nch_s = s_pad // (NW * CH)
    nch_e = e_pad // (NW * CH)
    et = e_pad // NW

    @functools.partial(
        pl.kernel,
        out_type=(
            jax.ShapeDtypeStruct((n_pad, EMB), f32),
            jax.ShapeDtypeStruct((s_pad, EMB), f32),
            jax.ShapeDtypeStruct((s_pad,), i32),
            jax.ShapeDtypeStruct((NW, (n_deg + 16) // 16, 16), f32),
            jax.ShapeDtypeStruct((e_pad // CH, CH), i32),
            jax.ShapeDtypeStruct((e_pad // CH, CH), i32),
        ),
        mesh=_mesh(),
        compiler_params=_SC_PARAMS,
        scratch_types=[
            pltpu.VMEM((CH,), i32),
            pltpu.VMEM((CH, EMB), f32),
            pltpu.VMEM((CH,), i32),
            pltpu.VMEM((CH,), i32),
            pltpu.VMEM((16, 16), f32),
            pltpu.VMEM(((n_deg + 16) // 16, 16), f32),
            pltpu.VMEM((4, CH), i32),
            pltpu.VMEM((CH,), i32),
            pltpu.VMEM((CH,), i32),
            pltpu.SemaphoreType.DMA,
        ],
    )
    def sc_prep(t1_hbm, t2_hbm, xg_hbm, stg_hbm, sb_hbm, s0_hbm, xsl_hbm,
                idx4_hbm, zdeg_hbm,
                h0_hbm, st2_hbm, sidx_hbm, degpart_hbm, gidx_hbm, zidx_hbm,
                idxb, rowb, s0b, sidxb, xslv, degv, islot, gb, sb2, sem):
        w = _wid()
        ii = lax.iota(i32, 16)
        for k in range(nch_n):
            base = w * (nch_n * CH) + k * CH
            pltpu.sync_copy(xg_hbm.at[pl.ds(base, CH)], idxb)
            pltpu.async_copy(t1_hbm.at[idxb], rowb, sem).wait()
            pltpu.sync_copy(rowb, h0_hbm.at[pl.ds(base, CH)])
        for k in range(nch_s):
            base = w * (nch_s * CH) + k * CH
            pltpu.sync_copy(stg_hbm.at[pl.ds(base, CH)], idxb)
            pltpu.async_copy(t2_hbm.at[idxb], rowb, sem).wait()
            pltpu.sync_copy(rowb, st2_hbm.at[pl.ds(base, CH)])
        pltpu.sync_copy(xsl_hbm, xslv)
        for k in range(nch_s):
            base = w * (nch_s * CH) + k * CH
            pltpu.sync_copy(sb_hbm.at[pl.ds(base, CH)], idxb)
            pltpu.sync_copy(s0_hbm.at[pl.ds(base, CH)], s0b)
            for g in range(CH // 16):
                sb = idxb[pl.ds(g * 16, 16)]
                sv = plsc.bitcast(plsc.load_gather(
                    xslv, [lax.shift_right_logical(sb, 4),
                           lax.bitwise_and(sb, 15)]), i32)
                sidxb[pl.ds(g * 16, 16)] = sv + s0b[pl.ds(g * 16, 16)]
            pltpu.sync_copy(sidxb, sidx_hbm.at[pl.ds(base, CH)])
        pltpu.sync_copy(zdeg_hbm, degv)
        ones16 = jnp.full((16,), 1.0, dtype=f32)
        nbv = jnp.full((16,), n_bond, dtype=i32)
        for k in range(nch_e):
            ct = w * nch_e + k
            pltpu.sync_copy(idx4_hbm.at[ct], islot)
            for g in range(CH // 16):
                sv = islot[0, pl.ds(g * 16, 16)]
                dvec = islot[1, pl.ds(g * 16, 16)]
                e0v = islot[2, pl.ds(g * 16, 16)]
                e1v = islot[3, pl.ds(g * 16, 16)]
                plsc.addupdate_scatter(
                    degv, [lax.shift_right_logical(dvec, 4),
                           lax.bitwise_and(dvec, 15)], ones16)
                gb[pl.ds(g * 16, 16)] = sv * nbv + e0v
                sb2[pl.ds(g * 16, 16)] = dvec * nbv + e1v
            pltpu.sync_copy(gb, gidx_hbm.at[ct])
            pltpu.sync_copy(sb2, zidx_hbm.at[ct])
        pltpu.sync_copy(degv, degpart_hbm.at[w])

    return sc_prep


def _make_sc_edge(n, n_agg, e_pad, n_bond):
    et = e_pad // NW
    nch_t = et // CH
    ztot = _ceil_to(n_agg * n_bond, NS * 8)
    zlen = ztot // NS

    @functools.partial(
        pl.kernel,
        out_type=jax.ShapeDtypeStruct((NC, ztot), f32),
        mesh=_mesh(),
        compiler_params=_SC_PARAMS,
        scratch_types=[
            pltpu.VMEM((nch_t, CH), i32),
            pltpu.VMEM((nch_t, CH), i32),
            pltpu.VMEM((nch_t, CH), f32),
            pltpu.VMEM_SHARED((ztot,), f32),
            pltpu.SemaphoreType.DMA,
            pltpu.SemaphoreType.DMA,
        ],
    )
    def sc_edge(p_hbm, gidx_hbm, zidx_hbm, z_hbm,
                agg_hbm,
                gix, six, sbuf, zz, semg, sems):
        cid = lax.axis_index("c")
        sid = lax.axis_index("s")
        w = sid * NC + cid
        r0 = w * nch_t
        pltpu.async_copy(gidx_hbm.at[pl.ds(r0, nch_t)], gix, semg)
        pltpu.async_copy(zidx_hbm.at[pl.ds(r0, nch_t)], six, semg)
        pltpu.sync_copy(z_hbm.at[pl.ds(sid * zlen, zlen)],
                        zz.at[pl.ds(sid * zlen, zlen)])
        pltpu.make_async_copy(gidx_hbm.at[pl.ds(r0, nch_t)], gix, semg).wait()
        pltpu.make_async_copy(gidx_hbm.at[pl.ds(r0, nch_t)], six, semg).wait()
        plsc.subcore_barrier()
        for k in range(nch_t):
            pltpu.async_copy(p_hbm.at[gix.at[k]], sbuf.at[k], semg)
        for k in range(nch_t):
            pltpu.make_async_copy(
                p_hbm.at[pl.ds(0, CH)], sbuf.at[k], semg).wait()
        for k in range(nch_t):
            pltpu.async_copy(sbuf.at[k], zz.at[six.at[k]], sems, add=True)
        for k in range(nch_t):
            pltpu.make_async_copy(
                sbuf.at[k], zz.at[pl.ds(0, CH)], sems).wait()
        plsc.subcore_barrier()
        pltpu.sync_copy(zz.at[pl.ds(sid * zlen, zlen)],
                        agg_hbm.at[cid, pl.ds(sid * zlen, zlen)])

    return sc_edge


def _make_sc_hgather(n, s_pad):
    nch_s = s_pad // (NW * CH)

    @functools.partial(
        pl.kernel,
        out_type=jax.ShapeDtypeStruct((s_pad, EMB), f32),
        mesh=_mesh(),
        compiler_params=_SC_PARAMS,
        scratch_types=[
            pltpu.VMEM((CH,), i32),
            pltpu.VMEM((CH, EMB), f32),
            pltpu.SemaphoreType.DMA,
        ],
    )
    def sc_hgather(h_hbm, sidx_hbm, out_hbm, idxb, rowb, sem):
        w = _wid()
        for k in range(nch_s):
            base = w * (nch_s * CH) + k * CH
            pltpu.sync_copy(sidx_hbm.at[pl.ds(base, CH)], idxb)
            pltpu.async_copy(h_hbm.at[idxb], rowb, sem).wait()
            pltpu.sync_copy(rowb, out_hbm.at[pl.ds(base, CH)])

    return sc_hgather


def kernel(x, edge_index, edge_attrs, stem_types, stems, stems_batch, batch,
           x_slices, frag_emb_w, stem_emb_w, bond_emb_w, conv_root_w,
           conv_bias, gru_w_ih, gru_w_hh, gru_b_ih, gru_b_hh, f2e_w1, f2e_b1,
           f2e_w2, f2e_b2, s2p_w1, s2p_b1, s2p_w2, s2p_b2, s2p_w3, s2p_b3,
           g2p_w1, g2p_b1, g2p_w2, g2p_b2):
    n = x.shape[0]
    e = edge_index.shape[1]
    s = stem_types.shape[0]
    bn = x_slices.shape[0] - 1
    n_bond = bond_emb_w.shape[0]
    npg = n // bn

    e_pad = _ceil_to(e, NW * CH)
    n_pad = _ceil_to(n, NW * CH)
    s_pad = _ceil_to(s, NW * CH)
    n_agg = _ceil_to(n + 16, NS)

    srcp = jnp.concatenate([edge_index[0], jnp.zeros((e_pad - e,), i32)])
    dstp = jnp.concatenate([edge_index[1], jnp.full((e_pad - e,), n, i32)])
    e0p = jnp.concatenate([edge_attrs[:, 0], jnp.zeros((e_pad - e,), i32)])
    e1p = jnp.concatenate([edge_attrs[:, 1], jnp.zeros((e_pad - e,), i32)])
    xgp = jnp.concatenate([x, jnp.zeros((n_pad - n,), i32)])
    stgp = jnp.concatenate([stem_types, jnp.zeros((s_pad - s,), i32)])
    sbp = jnp.concatenate([stems_batch, jnp.zeros((s_pad - s,), i32)])
    s0p = jnp.concatenate([stems[:, 0], jnp.zeros((s_pad - s,), i32)])
    xslp = jnp.concatenate(
        [x_slices, jnp.zeros((256 - x_slices.shape[0],), i32)]).reshape(16, 16)
    xslp = lax.bitcast_convert_type(xslp, f32)
    ztot = _ceil_to(n_agg * n_bond, NS * 8)
    zeros_z = jnp.zeros((ztot,), f32)
    zeros_deg = jnp.zeros(((n + 16) // 16, 16), f32)

    w1at = s2p_w1[:, :EMB].T
    w1bt = s2p_w1[:, EMB:].T
    wirt, wizt, wint = (gru_w_ih[0:EMB].T, gru_w_ih[EMB:2 * EMB].T,
                        gru_w_ih[2 * EMB:].T)
    whrt, whzt, whnt = (gru_w_hh[0:EMB].T, gru_w_hh[EMB:2 * EMB].T,
                        gru_w_hh[2 * EMB:].T)
    bir, biz, bin_ = (gru_b_ih[0:EMB].reshape(1, EMB),
                      gru_b_ih[EMB:2 * EMB].reshape(1, EMB),
                      gru_b_ih[2 * EMB:].reshape(1, EMB))
    bhr, bhz, bhn = (gru_b_hh[0:EMB].reshape(1, EMB),
                     gru_b_hh[EMB:2 * EMB].reshape(1, EMB),
                     gru_b_hh[2 * EMB:].reshape(1, EMB))

    nf = frag_emb_w.shape[0]
    nst = stem_emb_w.shape[0]
    t1, t2 = pl.pallas_call(
        _tc_prep_body,
        out_shape=(jax.ShapeDtypeStruct((nf, EMB), f32),
                   jax.ShapeDtypeStruct((nst, EMB), f32)),
    )(frag_emb_w, f2e_w1.T, f2e_b1.reshape(1, EMB), f2e_w2.T,
      f2e_b2.reshape(1, EMB), stem_emb_w, w1bt)

    idx4 = jnp.stack([srcp, dstp, e0p, e1p]).reshape(
        4, e_pad // CH, CH).transpose(1, 0, 2)
    sc_prep = _make_sc_prep(n_pad, s_pad, n, e_pad, n_bond)
    h0p, st2p, sidxp, degpart, gidx2, zidx2 = sc_prep(
        t1, t2, xgp, stgp, sbp, s0p, xslp, idx4, zeros_deg)
    h = h0p[:n]

    sc_edge = _make_sc_edge(n, n_agg, e_pad, n_bond)
    rb = 1000
    grid = n // rb
    bondt = bond_emb_w.T
    p = pl.pallas_call(
        _tc_p0_body,
        grid=(grid,),
        in_specs=[
            pl.BlockSpec((rb, EMB), lambda i: (i, 0)),
            pl.BlockSpec((EMB, n_bond), lambda i: (0, 0)),
        ],
        out_specs=pl.BlockSpec((rb, n_bond), lambda i: (i, 0)),
        out_shape=jax.ShapeDtypeStruct((n, n_bond), f32),
    )(h, bondt)
    gru_call = pl.pallas_call(
        _tc_gru_body,
        grid=(grid,),
        in_specs=[
            pl.BlockSpec((NC, rb, n_bond), lambda i: (0, i, 0)),
            pl.BlockSpec((rb, EMB), lambda i: (i, 0)),
            pl.BlockSpec((rb, NW), lambda i: (i, 0)),
            pl.BlockSpec((n_bond, EMB), lambda i: (0, 0)),
            pl.BlockSpec((EMB, n_bond), lambda i: (0, 0)),
        ] + [pl.BlockSpec((EMB, EMB), lambda i: (0, 0))] * 7
          + [pl.BlockSpec((1, EMB), lambda i: (0, 0))] * 7,
        out_specs=[pl.BlockSpec((rb, EMB), lambda i: (i, 0)),
                   pl.BlockSpec((rb, n_bond), lambda i: (i, 0))],
        out_shape=(jax.ShapeDtypeStruct((n, EMB), f32),
                   jax.ShapeDtypeStruct((n, n_bond), f32)),
    )
    degpt = degpart.reshape(NW, -1)[:, :n].T
    for _ in range(NUM_CONV_STEPS):
        aggz = sc_edge(p.reshape(-1), gidx2, zidx2, zeros_z)
        aggz = aggz[:, :n * n_bond].reshape(NC, n, n_bond)
        h, p = gru_call(aggz, h, degpt, bond_emb_w, bondt,
                        conv_root_w.T, wirt, wizt, wint,
                        whrt, whzt, whnt, conv_bias.reshape(1, EMB),
                        bir, biz, bin_, bhr, bhz, bhn)

    sc_hgather = _make_sc_hgather(n, s_pad)
    hsp = sc_hgather(h, sidxp)

    ops = s2p_w3.shape[0]
    sb_rows = 2000
    stem_preds = pl.pallas_call(
        _tc_stems_body,
        grid=(s // sb_rows,),
        in_specs=[
            pl.BlockSpec((sb_rows, EMB), lambda i: (i, 0)),
            pl.BlockSpec((sb_rows, EMB), lambda i: (i, 0)),
            pl.BlockSpec((EMB, EMB), lambda i: (0, 0)),
            pl.BlockSpec((1, EMB), lambda i: (0, 0)),
            pl.BlockSpec((EMB, EMB), lambda i: (0, 0)),
            pl.BlockSpec((1, EMB), lambda i: (0, 0)),
            pl.BlockSpec((EMB, ops), lambda i: (0, 0)),
            pl.BlockSpec((1, ops), lambda i: (0, 0)),
        ],
        out_specs=pl.BlockSpec((sb_rows, ops), lambda i: (i, 0)),
        out_shape=jax.ShapeDtypeStruct((s, ops), f32),
    )(hsp[:s], st2p[:s], w1at, s2p_b1.reshape(1, EMB), s2p_w2.T,
      s2p_b2.reshape(1, EMB), s2p_w3.T, s2p_b3.reshape(1, ops))

    opm = g2p_w2.shape[0]
    mol_preds = pl.pallas_call(
        _tc_pool_body,
        out_shape=jax.ShapeDtypeStruct((bn, opm), f32),
    )(h.reshape(bn, npg, EMB), g2p_w1.T, g2p_b1.reshape(1, EMB), g2p_w2.T,
      g2p_b2.reshape(1, opm))

    return (mol_preds, stem_preds)


# speedup vs baseline: 10.6716x; 1.3444x over previous
import functools

import jax
import jax.numpy as jnp
from jax import lax
from jax.experimental import pallas as pl
from jax.experimental.pallas import tpu as pltpu
from jax.experimental.pallas import tpu_sc as plsc

EMB = 32
NC = 2
NS = 16
NW = NC * NS
CH = 128
NUM_CONV_STEPS = 4

f32 = jnp.float32
i32 = jnp.int32


def _ceil_to(v, m):
    return (v + m - 1) // m * m


def _lrelu(v):
    return jnp.where(v >= 0, v, 0.01 * v)


def _tc_prep_body(frag_ref, w1t_ref, b1_ref, w2t_ref, b2_ref, stem_ref,
                  w1bt_ref, t1_ref, t2_ref):
    z = _lrelu(jnp.dot(frag_ref[...], w1t_ref[...],
                       preferred_element_type=f32) + b1_ref[...])
    t1_ref[...] = jnp.dot(z, w2t_ref[...], preferred_element_type=f32) + b2_ref[...]
    t2_ref[...] = jnp.dot(stem_ref[...], w1bt_ref[...], preferred_element_type=f32)


def _tc_gru_body(aggz_ref, h_ref, degp_ref, bond_ref, bondt_ref, wrt_ref,
                 wir_ref, wiz_ref, win_ref, whr_ref, whz_ref, whn_ref,
                 cb_ref, bir_ref, biz_ref, bin_ref, bhr_ref, bhz_ref, bhn_ref,
                 out_ref, p_ref):
    deg = jnp.sum(degp_ref[...], axis=1)
    deginv = 1.0 / jnp.maximum(deg, 1.0)
    hb = h_ref[...]
    zsum = aggz_ref[0] + aggz_ref[1]
    agg = jnp.dot(zsum, bond_ref[...],
                  preferred_element_type=f32) * deginv[:, None]
    conv = agg + jnp.dot(hb, wrt_ref[...], preferred_element_type=f32) + cb_ref[...]
    m = _lrelu(conv)
    gir = jnp.dot(m, wir_ref[...], preferred_element_type=f32) + bir_ref[...]
    giz = jnp.dot(m, wiz_ref[...], preferred_element_type=f32) + biz_ref[...]
    gin = jnp.dot(m, win_ref[...], preferred_element_type=f32) + bin_ref[...]
    ghr = jnp.dot(hb, whr_ref[...], preferred_element_type=f32) + bhr_ref[...]
    ghz = jnp.dot(hb, whz_ref[...], preferred_element_type=f32) + bhz_ref[...]
    ghn = jnp.dot(hb, whn_ref[...], preferred_element_type=f32) + bhn_ref[...]
    r = jax.nn.sigmoid(gir + ghr)
    z = jax.nn.sigmoid(giz + ghz)
    n = jnp.tanh(gin + r * ghn)
    hn = (1.0 - z) * n + z * hb
    out_ref[...] = hn
    p_ref[...] = jnp.dot(hn, bondt_ref[...], preferred_element_type=f32)


def _tc_p0_body(h_ref, bondt_ref, out_ref):
    out_ref[...] = jnp.dot(h_ref[...], bondt_ref[...],
                           preferred_element_type=f32)


def _tc_stems_body(hs_ref, st2_ref, w1at_ref, b1_ref, w2t_ref, b2_ref,
                   w3t_ref, b3_ref, out_ref):
    sp = _lrelu(jnp.dot(hs_ref[...], w1at_ref[...], preferred_element_type=f32)
                + st2_ref[...] + b1_ref[...])
    sp = _lrelu(jnp.dot(sp, w2t_ref[...], preferred_element_type=f32) + b2_ref[...])
    out_ref[...] = jnp.dot(sp, w3t_ref[...], preferred_element_type=f32) + b3_ref[...]


def _tc_pool_body(h3_ref, w1t_ref, b1_ref, w2t_ref, b2_ref, out_ref):
    pooled = jnp.mean(h3_ref[...], axis=1)
    mp = _lrelu(jnp.dot(pooled, w1t_ref[...], preferred_element_type=f32) + b1_ref[...])
    out_ref[...] = jnp.dot(mp, w2t_ref[...], preferred_element_type=f32) + b2_ref[...]


def _mesh():
    return plsc.VectorSubcoreMesh(core_axis_name="c", subcore_axis_name="s")


_SC_PARAMS = pltpu.CompilerParams(needs_layout_passes=False,
                                  use_tc_tiling_on_sc=False)


def _wid():
    return lax.axis_index("s") * NC + lax.axis_index("c")


def _make_sc_prep(n_pad, s_pad, n_deg, e_pad, n_bond):
    nch_n = n_pad // (NW * CH)
    nch_s = s_pad // (NW * CH)
    nch_e = e_pad // (NW * CH)

    @functools.partial(
        pl.kernel,
        out_type=(
            jax.ShapeDtypeStruct((n_pad, EMB), f32),
            jax.ShapeDtypeStruct((s_pad, EMB), f32),
            jax.ShapeDtypeStruct((s_pad // CH, CH), i32),
            jax.ShapeDtypeStruct((NW, (n_deg + 16) // 16, 16), f32),
            jax.ShapeDtypeStruct((e_pad // CH, CH), i32),
            jax.ShapeDtypeStruct((e_pad // CH, CH), i32),
        ),
        mesh=_mesh(),
        compiler_params=_SC_PARAMS,
        scratch_types=[
            pltpu.VMEM((nch_s, CH), i32),
            pltpu.VMEM((nch_s * CH, EMB), f32),
            pltpu.VMEM((nch_s, CH), i32),
            pltpu.VMEM((nch_s, CH), i32),
            pltpu.VMEM((16, 16), f32),
            pltpu.VMEM(((n_deg + 16) // 16, 16), f32),
            pltpu.VMEM((nch_e, CH), i32),
            pltpu.VMEM((nch_e, CH), i32),
            pltpu.VMEM((nch_e, CH), i32),
            pltpu.VMEM((nch_e, CH), i32),
            pltpu.VMEM((nch_e, CH), i32),
            pltpu.VMEM((nch_e, CH), i32),
            pltpu.SemaphoreType.DMA,
        ],
    )
    def sc_prep(t1_hbm, t2_hbm, xg_hbm, stg_hbm, sb_hbm, s0_hbm, xsl_hbm,
                src_hbm, dst_hbm, e0_hbm, e1_hbm, zdeg_hbm,
                h0_hbm, st2_hbm, sidx_hbm, degpart_hbm, gidx_hbm, zidx_hbm,
                idxb, rowb, s0b, sidxb, xslv, degv,
                sv, dv, e0v, e1v, gb, sb2, sem):
        w = _wid()
        pltpu.sync_copy(xg_hbm.at[pl.ds(w * nch_n, nch_n)],
                        idxb.at[pl.ds(0, nch_n)])
        for k in range(nch_n):
            pltpu.async_copy(t1_hbm.at[idxb.at[k]],
                             rowb.at[pl.ds(k * CH, CH)], sem)
        for k in range(nch_n):
            pltpu.make_async_copy(t1_hbm.at[pl.ds(0, CH)],
                                  rowb.at[pl.ds(k * CH, CH)], sem).wait()
        pltpu.sync_copy(rowb.at[pl.ds(0, nch_n * CH)],
                        h0_hbm.at[pl.ds(w * nch_n * CH, nch_n * CH)])
        pltpu.sync_copy(stg_hbm.at[pl.ds(w * nch_s, nch_s)], idxb)
        for k in range(nch_s):
            pltpu.async_copy(t2_hbm.at[idxb.at[k]],
                             rowb.at[pl.ds(k * CH, CH)], sem)
        for k in range(nch_s):
            pltpu.make_async_copy(t2_hbm.at[pl.ds(0, CH)],
                                  rowb.at[pl.ds(k * CH, CH)], sem).wait()
        pltpu.sync_copy(rowb, st2_hbm.at[pl.ds(w * nch_s * CH, nch_s * CH)])
        pltpu.sync_copy(xsl_hbm, xslv)
        pltpu.sync_copy(sb_hbm.at[pl.ds(w * nch_s, nch_s)], idxb)
        pltpu.sync_copy(s0_hbm.at[pl.ds(w * nch_s, nch_s)], s0b)
        for k in range(nch_s):
            for g in range(CH // 16):
                sb = idxb[k, pl.ds(g * 16, 16)]
                svv = plsc.bitcast(plsc.load_gather(
                    xslv, [lax.shift_right_logical(sb, 4),
                           lax.bitwise_and(sb, 15)]), i32)
                sidxb[k, pl.ds(g * 16, 16)] = svv + s0b[k, pl.ds(g * 16, 16)]
        pltpu.sync_copy(sidxb, sidx_hbm.at[pl.ds(w * nch_s, nch_s)])
        pltpu.sync_copy(zdeg_hbm, degv)
        pltpu.sync_copy(src_hbm.at[pl.ds(w * nch_e, nch_e)], sv)
        pltpu.sync_copy(dst_hbm.at[pl.ds(w * nch_e, nch_e)], dv)
        pltpu.sync_copy(e0_hbm.at[pl.ds(w * nch_e, nch_e)], e0v)
        pltpu.sync_copy(e1_hbm.at[pl.ds(w * nch_e, nch_e)], e1v)
        ones16 = jnp.full((16,), 1.0, dtype=f32)
        nbv = jnp.full((16,), n_bond, dtype=i32)
        for k in range(nch_e):
            for g in range(CH // 16):
                svec = sv[k, pl.ds(g * 16, 16)]
                dvec = dv[k, pl.ds(g * 16, 16)]
                e0g = e0v[k, pl.ds(g * 16, 16)]
                e1g = e1v[k, pl.ds(g * 16, 16)]
                plsc.addupdate_scatter(
                    degv, [lax.shift_right_logical(dvec, 4),
                           lax.bitwise_and(dvec, 15)], ones16)
                gb[k, pl.ds(g * 16, 16)] = svec * nbv + e0g
                sb2[k, pl.ds(g * 16, 16)] = dvec * nbv + e1g
        pltpu.sync_copy(gb, gidx_hbm.at[pl.ds(w * nch_e, nch_e)])
        pltpu.sync_copy(sb2, zidx_hbm.at[pl.ds(w * nch_e, nch_e)])
        pltpu.sync_copy(degv, degpart_hbm.at[w])

    return sc_prep


def _make_sc_edge(n, n_agg, e_pad, n_bond):
    et = e_pad // NW
    nch_t = et // CH
    ztot = _ceil_to(n_agg * n_bond, NS * 8)
    zlen = ztot // NS

    @functools.partial(
        pl.kernel,
        out_type=jax.ShapeDtypeStruct((NC, ztot), f32),
        mesh=_mesh(),
        compiler_params=_SC_PARAMS,
        scratch_types=[
            pltpu.VMEM((nch_t, CH), i32),
            pltpu.VMEM((nch_t, CH), i32),
            pltpu.VMEM((nch_t, CH), f32),
            pltpu.VMEM_SHARED((ztot,), f32),
            pltpu.SemaphoreType.DMA,
            pltpu.SemaphoreType.DMA,
        ],
    )
    def sc_edge(p_hbm, gidx_hbm, zidx_hbm, z_hbm,
                agg_hbm,
                gix, six, sbuf, zz, semg, sems):
        cid = lax.axis_index("c")
        sid = lax.axis_index("s")
        w = sid * NC + cid
        r0 = w * nch_t
        pltpu.async_copy(gidx_hbm.at[pl.ds(r0, nch_t)], gix, semg)
        pltpu.async_copy(zidx_hbm.at[pl.ds(r0, nch_t)], six, semg)
        pltpu.sync_copy(z_hbm.at[pl.ds(sid * zlen, zlen)],
                        zz.at[pl.ds(sid * zlen, zlen)])
        pltpu.make_async_copy(gidx_hbm.at[pl.ds(r0, nch_t)], gix, semg).wait()
        pltpu.make_async_copy(gidx_hbm.at[pl.ds(r0, nch_t)], six, semg).wait()
        plsc.subcore_barrier()
        for k in range(nch_t):
            pltpu.async_copy(p_hbm.at[gix.at[k]], sbuf.at[k], semg)
        for k in range(nch_t):
            pltpu.make_async_copy(
                p_hbm.at[pl.ds(0, CH)], sbuf.at[k], semg).wait()
        for k in range(nch_t):
            pltpu.async_copy(sbuf.at[k], zz.at[six.at[k]], sems, add=True)
        for k in range(nch_t):
            pltpu.make_async_copy(
                sbuf.at[k], zz.at[pl.ds(0, CH)], sems).wait()
        plsc.subcore_barrier()
        pltpu.sync_copy(zz.at[pl.ds(sid * zlen, zlen)],
                        agg_hbm.at[cid, pl.ds(sid * zlen, zlen)])

    return sc_edge


def _make_sc_hgather(n, s_pad):
    nch_s = s_pad // (NW * CH)

    @functools.partial(
        pl.kernel,
        out_type=jax.ShapeDtypeStruct((s_pad, EMB), f32),
        mesh=_mesh(),
        compiler_params=_SC_PARAMS,
        scratch_types=[
            pltpu.VMEM((nch_s, CH), i32),
            pltpu.VMEM((nch_s * CH, EMB), f32),
            pltpu.SemaphoreType.DMA,
        ],
    )
    def sc_hgather(h_hbm, sidx_hbm, out_hbm, idxb, rowb, sem):
        w = _wid()
        pltpu.sync_copy(sidx_hbm.at[pl.ds(w * nch_s, nch_s)], idxb)
        for k in range(nch_s):
            pltpu.async_copy(h_hbm.at[idxb.at[k]],
                             rowb.at[pl.ds(k * CH, CH)], sem)
        for k in range(nch_s):
            pltpu.make_async_copy(h_hbm.at[pl.ds(0, CH)],
                                  rowb.at[pl.ds(k * CH, CH)], sem).wait()
        pltpu.sync_copy(rowb, out_hbm.at[pl.ds(w * nch_s * CH, nch_s * CH)])

    return sc_hgather


def kernel(x, edge_index, edge_attrs, stem_types, stems, stems_batch, batch,
           x_slices, frag_emb_w, stem_emb_w, bond_emb_w, conv_root_w,
           conv_bias, gru_w_ih, gru_w_hh, gru_b_ih, gru_b_hh, f2e_w1, f2e_b1,
           f2e_w2, f2e_b2, s2p_w1, s2p_b1, s2p_w2, s2p_b2, s2p_w3, s2p_b3,
           g2p_w1, g2p_b1, g2p_w2, g2p_b2):
    n = x.shape[0]
    e = edge_index.shape[1]
    s = stem_types.shape[0]
    bn = x_slices.shape[0] - 1
    n_bond = bond_emb_w.shape[0]
    npg = n // bn

    e_pad = _ceil_to(e, NW * CH)
    n_pad = _ceil_to(n, NW * CH)
    s_pad = _ceil_to(s, NW * CH)
    n_agg = _ceil_to(n + 16, NS * 8)

    srcp = jnp.concatenate([edge_index[0], jnp.zeros((e_pad - e,), i32)])
    dstp = jnp.concatenate([edge_index[1], jnp.full((e_pad - e,), n, i32)])
    e0p = jnp.concatenate([edge_attrs[:, 0], jnp.zeros((e_pad - e,), i32)])
    e1p = jnp.concatenate([edge_attrs[:, 1], jnp.zeros((e_pad - e,), i32)])
    xgp = jnp.concatenate([x, jnp.zeros((n_pad - n,), i32)])
    stgp = jnp.concatenate([stem_types, jnp.zeros((s_pad - s,), i32)])
    sbp = jnp.concatenate([stems_batch, jnp.zeros((s_pad - s,), i32)])
    s0p = jnp.concatenate([stems[:, 0], jnp.zeros((s_pad - s,), i32)])
    xslp = jnp.concatenate(
        [x_slices, jnp.zeros((256 - x_slices.shape[0],), i32)]).reshape(16, 16)
    xslp = lax.bitcast_convert_type(xslp, f32)
    zeros_z = jnp.zeros((_ceil_to(n_agg * n_bond, NS * 8),), f32)
    zeros_deg = jnp.zeros(((n + 16) // 16, 16), f32)

    w1at = s2p_w1[:, :EMB].T
    w1bt = s2p_w1[:, EMB:].T
    wirt, wizt, wint = (gru_w_ih[0:EMB].T, gru_w_ih[EMB:2 * EMB].T,
                        gru_w_ih[2 * EMB:].T)
    whrt, whzt, whnt = (gru_w_hh[0:EMB].T, gru_w_hh[EMB:2 * EMB].T,
                        gru_w_hh[2 * EMB:].T)
    bir, biz, bin_ = (gru_b_ih[0:EMB].reshape(1, EMB),
                      gru_b_ih[EMB:2 * EMB].reshape(1, EMB),
                      gru_b_ih[2 * EMB:].reshape(1, EMB))
    bhr, bhz, bhn = (gru_b_hh[0:EMB].reshape(1, EMB),
                     gru_b_hh[EMB:2 * EMB].reshape(1, EMB),
                     gru_b_hh[2 * EMB:].reshape(1, EMB))

    nf = frag_emb_w.shape[0]
    nst = stem_emb_w.shape[0]
    t1, t2 = pl.pallas_call(
        _tc_prep_body,
        out_shape=(jax.ShapeDtypeStruct((nf, EMB), f32),
                   jax.ShapeDtypeStruct((nst, EMB), f32)),
    )(frag_emb_w, f2e_w1.T, f2e_b1.reshape(1, EMB), f2e_w2.T,
      f2e_b2.reshape(1, EMB), stem_emb_w, w1bt)

    sc_prep = _make_sc_prep(n_pad, s_pad, n, e_pad, n_bond)
    h0p, st2p, sidxp, degpart, gidx2, zidx2 = sc_prep(
        t1, t2, xgp.reshape(-1, CH), stgp.reshape(-1, CH),
        sbp.reshape(-1, CH), s0p.reshape(-1, CH), xslp,
        srcp.reshape(-1, CH), dstp.reshape(-1, CH),
        e0p.reshape(-1, CH), e1p.reshape(-1, CH), zeros_deg)
    h = h0p[:n]

    sc_edge = _make_sc_edge(n, n_agg, e_pad, n_bond)
    rb = 1000
    grid = n // rb
    bondt = bond_emb_w.T
    p = pl.pallas_call(
        _tc_p0_body,
        grid=(grid,),
        in_specs=[
            pl.BlockSpec((rb, EMB), lambda i: (i, 0)),
            pl.BlockSpec((EMB, n_bond), lambda i: (0, 0)),
        ],
        out_specs=pl.BlockSpec((rb, n_bond), lambda i: (i, 0)),
        out_shape=jax.ShapeDtypeStruct((n, n_bond), f32),
    )(h, bondt)
    gru_call = pl.pallas_call(
        _tc_gru_body,
        grid=(grid,),
        in_specs=[
            pl.BlockSpec((NC, rb, n_bond), lambda i: (0, i, 0)),
            pl.BlockSpec((rb, EMB), lambda i: (i, 0)),
            pl.BlockSpec((rb, NW), lambda i: (i, 0)),
            pl.BlockSpec((n_bond, EMB), lambda i: (0, 0)),
            pl.BlockSpec((EMB, n_bond), lambda i: (0, 0)),
        ] + [pl.BlockSpec((EMB, EMB), lambda i: (0, 0))] * 7
          + [pl.BlockSpec((1, EMB), lambda i: (0, 0))] * 7,
        out_specs=[pl.BlockSpec((rb, EMB), lambda i: (i, 0)),
                   pl.BlockSpec((rb, n_bond), lambda i: (i, 0))],
        out_shape=(jax.ShapeDtypeStruct((n, EMB), f32),
                   jax.ShapeDtypeStruct((n, n_bond), f32)),
    )
    degpt = degpart.reshape(NW, -1)[:, :n].T
    for _ in range(NUM_CONV_STEPS):
        aggz = sc_edge(p.reshape(-1), gidx2, zidx2, zeros_z)
        aggz = aggz.reshape(NC, n_agg, n_bond)
        h, p = gru_call(aggz, h, degpt, bond_emb_w, bondt,
                        conv_root_w.T, wirt, wizt, wint,
                        whrt, whzt, whnt, conv_bias.reshape(1, EMB),
                        bir, biz, bin_, bhr, bhz, bhn)

    sc_hgather = _make_sc_hgather(n, s_pad)
    hsp = sc_hgather(h, sidxp)

    ops = s2p_w3.shape[0]
    sb_rows = 2000
    stem_preds = pl.pallas_call(
        _tc_stems_body,
        grid=(s // sb_rows,),
        in_specs=[
            pl.BlockSpec((sb_rows, EMB), lambda i: (i, 0)),
            pl.BlockSpec((sb_rows, EMB), lambda i: (i, 0)),
            pl.BlockSpec((EMB, EMB), lambda i: (0, 0)),
            pl.BlockSpec((1, EMB), lambda i: (0, 0)),
            pl.BlockSpec((EMB, EMB), lambda i: (0, 0)),
            pl.BlockSpec((1, EMB), lambda i: (0, 0)),
            pl.BlockSpec((EMB, ops), lambda i: (0, 0)),
            pl.BlockSpec((1, ops), lambda i: (0, 0)),
        ],
        out_specs=pl.BlockSpec((sb_rows, ops), lambda i: (i, 0)),
        out_shape=jax.ShapeDtypeStruct((s, ops), f32),
    )(hsp[:s], st2p[:s], w1at, s2p_b1.reshape(1, EMB), s2p_w2.T,
      s2p_b2.reshape(1, EMB), s2p_w3.T, s2p_b3.reshape(1, ops))

    opm = g2p_w2.shape[0]
    mol_preds = pl.pallas_call(
        _tc_pool_body,
        out_shape=jax.ShapeDtypeStruct((bn, opm), f32),
    )(h.reshape(bn, npg, EMB), g2p_w1.T, g2p_b1.reshape(1, EMB), g2p_w2.T,
      g2p_b2.reshape(1, opm))

    return (mol_preds, stem_preds)


# speedup vs baseline: 10.9496x; 1.0260x over previous
import functools

import jax
import jax.numpy as jnp
from jax import lax
from jax.experimental import pallas as pl
from jax.experimental.pallas import tpu as pltpu
from jax.experimental.pallas import tpu_sc as plsc

EMB = 32
NC = 2
NS = 16
NW = NC * NS
CH = 128
NUM_CONV_STEPS = 4

f32 = jnp.float32
i32 = jnp.int32


def _ceil_to(v, m):
    return (v + m - 1) // m * m


def _lrelu(v):
    return jnp.where(v >= 0, v, 0.01 * v)


def _tc_prep_body(frag_ref, w1t_ref, b1_ref, w2t_ref, b2_ref, stem_ref,
                  w1bt_ref, t1_ref, t2_ref):
    z = _lrelu(jnp.dot(frag_ref[...], w1t_ref[...],
                       preferred_element_type=f32) + b1_ref[...])
    t1_ref[...] = jnp.dot(z, w2t_ref[...], preferred_element_type=f32) + b2_ref[...]
    t2_ref[...] = jnp.dot(stem_ref[...], w1bt_ref[...], preferred_element_type=f32)


def _tc_gru_body(aggz_ref, h_ref, degp_ref, bond_ref, bondt_ref, wrt_ref,
                 wir_ref, wiz_ref, win_ref, whr_ref, whz_ref, whn_ref,
                 cb_ref, bir_ref, biz_ref, bin_ref, bhr_ref, bhz_ref, bhn_ref,
                 out_ref, p_ref):
    deg = jnp.sum(degp_ref[...], axis=1)
    deginv = 1.0 / jnp.maximum(deg, 1.0)
    hb = h_ref[...]
    zsum = aggz_ref[0] + aggz_ref[1]
    agg = jnp.dot(zsum, bond_ref[...],
                  preferred_element_type=f32) * deginv[:, None]
    conv = agg + jnp.dot(hb, wrt_ref[...], preferred_element_type=f32) + cb_ref[...]
    m = _lrelu(conv)
    gir = jnp.dot(m, wir_ref[...], preferred_element_type=f32) + bir_ref[...]
    giz = jnp.dot(m, wiz_ref[...], preferred_element_type=f32) + biz_ref[...]
    gin = jnp.dot(m, win_ref[...], preferred_element_type=f32) + bin_ref[...]
    ghr = jnp.dot(hb, whr_ref[...], preferred_element_type=f32) + bhr_ref[...]
    ghz = jnp.dot(hb, whz_ref[...], preferred_element_type=f32) + bhz_ref[...]
    ghn = jnp.dot(hb, whn_ref[...], preferred_element_type=f32) + bhn_ref[...]
    r = jax.nn.sigmoid(gir + ghr)
    z = jax.nn.sigmoid(giz + ghz)
    n = jnp.tanh(gin + r * ghn)
    hn = (1.0 - z) * n + z * hb
    out_ref[...] = hn
    p_ref[...] = jnp.dot(hn, bondt_ref[...], preferred_element_type=f32)


def _tc_p0_body(h_ref, bondt_ref, out_ref):
    out_ref[...] = jnp.dot(h_ref[...], bondt_ref[...],
                           preferred_element_type=f32)


def _tc_stems_body(hs_ref, st2_ref, w1at_ref, b1_ref, w2t_ref, b2_ref,
                   w3t_ref, b3_ref, out_ref):
    sp = _lrelu(jnp.dot(hs_ref[...], w1at_ref[...], preferred_element_type=f32)
                + st2_ref[...] + b1_ref[...])
    sp = _lrelu(jnp.dot(sp, w2t_ref[...], preferred_element_type=f32) + b2_ref[...])
    out_ref[...] = jnp.dot(sp, w3t_ref[...], preferred_element_type=f32) + b3_ref[...]


def _tc_pool_body(h3_ref, w1t_ref, b1_ref, w2t_ref, b2_ref, out_ref):
    pooled = jnp.mean(h3_ref[...], axis=1)
    mp = _lrelu(jnp.dot(pooled, w1t_ref[...], preferred_element_type=f32) + b1_ref[...])
    out_ref[...] = jnp.dot(mp, w2t_ref[...], preferred_element_type=f32) + b2_ref[...]


def _mesh():
    return plsc.VectorSubcoreMesh(core_axis_name="c", subcore_axis_name="s")


_SC_PARAMS = pltpu.CompilerParams(needs_layout_passes=False,
                                  use_tc_tiling_on_sc=False)


def _wid():
    return lax.axis_index("s") * NC + lax.axis_index("c")


def _make_sc_prep(n_pad, s_pad, n_deg, e_pad, n_bond):
    nch_n = n_pad // (NW * CH)
    nch_s = s_pad // (NW * CH)
    nch_e = e_pad // (NW * CH)

    @functools.partial(
        pl.kernel,
        out_type=(
            jax.ShapeDtypeStruct((n_pad, EMB), f32),
            jax.ShapeDtypeStruct((s_pad, EMB), f32),
            jax.ShapeDtypeStruct((s_pad // CH, CH), i32),
            jax.ShapeDtypeStruct((NW, (n_deg + 16) // 16, 16), f32),
            jax.ShapeDtypeStruct((e_pad // CH, CH), i32),
            jax.ShapeDtypeStruct((e_pad // CH, CH), i32),
        ),
        mesh=_mesh(),
        compiler_params=_SC_PARAMS,
        scratch_types=[
            pltpu.VMEM((nch_n, CH), i32),
            pltpu.VMEM((nch_s, CH), i32),
            pltpu.VMEM((nch_s, CH), i32),
            pltpu.VMEM((nch_n * CH, EMB), f32),
            pltpu.VMEM((nch_s * CH, EMB), f32),
            pltpu.VMEM((nch_s, CH), i32),
            pltpu.VMEM((nch_s, CH), i32),
            pltpu.VMEM((16, 16), f32),
            pltpu.VMEM(((n_deg + 16) // 16, 16), f32),
            pltpu.VMEM((nch_e, CH), i32),
            pltpu.VMEM((nch_e, CH), i32),
            pltpu.VMEM((nch_e, CH), i32),
            pltpu.VMEM((nch_e, CH), i32),
            pltpu.VMEM((nch_e, CH), i32),
            pltpu.VMEM((nch_e, CH), i32),
            pltpu.SemaphoreType.DMA,
            pltpu.SemaphoreType.DMA,
            pltpu.SemaphoreType.DMA,
            pltpu.SemaphoreType.DMA,
        ],
    )
    def sc_prep(t1_hbm, t2_hbm, xg_hbm, stg_hbm, sb_hbm, s0_hbm, xsl_hbm,
                src_hbm, dst_hbm, e0_hbm, e1_hbm, zdeg_hbm,
                h0_hbm, st2_hbm, sidx_hbm, degpart_hbm, gidx_hbm, zidx_hbm,
                idxbn, idxbs, sbb, rowbn, rowbs, s0b, sidxb, xslv, degv,
                sv, dv, e0v, e1v, gb, sb2, sema, semb, semc, semg):
        w = _wid()
        pltpu.async_copy(xg_hbm.at[pl.ds(w * nch_n, nch_n)], idxbn, sema)
        pltpu.async_copy(stg_hbm.at[pl.ds(w * nch_s, nch_s)], idxbs, semb)
        pltpu.async_copy(sb_hbm.at[pl.ds(w * nch_s, nch_s)], sbb, semc)
        pltpu.async_copy(s0_hbm.at[pl.ds(w * nch_s, nch_s)], s0b, semc)
        pltpu.async_copy(xsl_hbm, xslv, semc)
        pltpu.async_copy(src_hbm.at[pl.ds(w * nch_e, nch_e)], sv, semc)
        pltpu.async_copy(dst_hbm.at[pl.ds(w * nch_e, nch_e)], dv, semc)
        pltpu.async_copy(e0_hbm.at[pl.ds(w * nch_e, nch_e)], e0v, semc)
        pltpu.async_copy(e1_hbm.at[pl.ds(w * nch_e, nch_e)], e1v, semc)
        pltpu.async_copy(zdeg_hbm, degv, semc)
        pltpu.make_async_copy(xg_hbm.at[pl.ds(0, nch_n)], idxbn, sema).wait()
        for k in range(nch_n):
            pltpu.async_copy(t1_hbm.at[idxbn.at[k]],
                             rowbn.at[pl.ds(k * CH, CH)], semg)
        pltpu.make_async_copy(stg_hbm.at[pl.ds(0, nch_s)], idxbs, semb).wait()
        for k in range(nch_s):
            pltpu.async_copy(t2_hbm.at[idxbs.at[k]],
                             rowbs.at[pl.ds(k * CH, CH)], semg)
        pltpu.make_async_copy(sb_hbm.at[pl.ds(0, nch_s)], sbb, semc).wait()
        pltpu.make_async_copy(sb_hbm.at[pl.ds(0, nch_s)], s0b, semc).wait()
        pltpu.make_async_copy(xsl_hbm, xslv, semc).wait()
        for buf in (sv, dv, e0v, e1v):
            pltpu.make_async_copy(src_hbm.at[pl.ds(0, nch_e)], buf,
                                  semc).wait()
        pltpu.make_async_copy(zdeg_hbm, degv, semc).wait()
        for k in range(nch_s):
            for g in range(CH // 16):
                sb = sbb[k, pl.ds(g * 16, 16)]
                svv = plsc.bitcast(plsc.load_gather(
                    xslv, [lax.shift_right_logical(sb, 4),
                           lax.bitwise_and(sb, 15)]), i32)
                sidxb[k, pl.ds(g * 16, 16)] = svv + s0b[k, pl.ds(g * 16, 16)]
        pltpu.sync_copy(sidxb, sidx_hbm.at[pl.ds(w * nch_s, nch_s)])
        ones16 = jnp.full((16,), 1.0, dtype=f32)
        nbv = jnp.full((16,), n_bond, dtype=i32)
        for k in range(nch_e):
            for g in range(CH // 16):
                svec = sv[k, pl.ds(g * 16, 16)]
                dvec = dv[k, pl.ds(g * 16, 16)]
                e0g = e0v[k, pl.ds(g * 16, 16)]
                e1g = e1v[k, pl.ds(g * 16, 16)]
                plsc.addupdate_scatter(
                    degv, [lax.shift_right_logical(dvec, 4),
                           lax.bitwise_and(dvec, 15)], ones16)
                gb[k, pl.ds(g * 16, 16)] = svec * nbv + e0g
                sb2[k, pl.ds(g * 16, 16)] = dvec * nbv + e1g
        pltpu.async_copy(gb, gidx_hbm.at[pl.ds(w * nch_e, nch_e)], semc)
        pltpu.async_copy(sb2, zidx_hbm.at[pl.ds(w * nch_e, nch_e)], semc)
        pltpu.async_copy(degv, degpart_hbm.at[w], semc)
        for k in range(nch_n + nch_s):
            pltpu.make_async_copy(t1_hbm.at[pl.ds(0, CH)],
                                  rowbn.at[pl.ds(0, CH)], semg).wait()
        pltpu.async_copy(rowbn, h0_hbm.at[pl.ds(w * nch_n * CH, nch_n * CH)],
                         semc)
        pltpu.async_copy(rowbs, st2_hbm.at[pl.ds(w * nch_s * CH, nch_s * CH)],
                         semc)
        pltpu.make_async_copy(gb, gidx_hbm.at[pl.ds(0, nch_e)], semc).wait()
        pltpu.make_async_copy(sb2, zidx_hbm.at[pl.ds(0, nch_e)], semc).wait()
        pltpu.make_async_copy(degv, degpart_hbm.at[0], semc).wait()
        pltpu.make_async_copy(
            rowbn, h0_hbm.at[pl.ds(0, nch_n * CH)], semc).wait()
        pltpu.make_async_copy(
            rowbs, st2_hbm.at[pl.ds(0, nch_s * CH)], semc).wait()

    return sc_prep


def _make_sc_edge(n, n_agg, e_pad, n_bond):
    et = e_pad // NW
    nch_t = et // CH
    ztot = _ceil_to(n_agg * n_bond, NS * 8)
    zlen = ztot // NS

    @functools.partial(
        pl.kernel,
        out_type=jax.ShapeDtypeStruct((NC, ztot), f32),
        mesh=_mesh(),
        compiler_params=_SC_PARAMS,
        scratch_types=[
            pltpu.VMEM((nch_t, CH), i32),
            pltpu.VMEM((nch_t, CH), i32),
            pltpu.VMEM((nch_t, CH), f32),
            pltpu.VMEM_SHARED((ztot,), f32),
            pltpu.SemaphoreType.DMA,
            pltpu.SemaphoreType.DMA,
        ],
    )
    def sc_edge(p_hbm, gidx_hbm, zidx_hbm, z_hbm,
                agg_hbm,
                gix, six, sbuf, zz, semg, sems):
        cid = lax.axis_index("c")
        sid = lax.axis_index("s")
        w = sid * NC + cid
        r0 = w * nch_t
        pltpu.async_copy(gidx_hbm.at[pl.ds(r0, nch_t)], gix, semg)
        pltpu.async_copy(zidx_hbm.at[pl.ds(r0, nch_t)], six, semg)
        pltpu.sync_copy(z_hbm.at[pl.ds(sid * zlen, zlen)],
                        zz.at[pl.ds(sid * zlen, zlen)])
        pltpu.make_async_copy(gidx_hbm.at[pl.ds(r0, nch_t)], gix, semg).wait()
        pltpu.make_async_copy(gidx_hbm.at[pl.ds(r0, nch_t)], six, semg).wait()
        plsc.subcore_barrier()
        for k in range(nch_t):
            pltpu.async_copy(p_hbm.at[gix.at[k]], sbuf.at[k], semg)
        for k in range(nch_t):
            pltpu.make_async_copy(
                p_hbm.at[pl.ds(0, CH)], sbuf.at[k], semg).wait()
        for k in range(nch_t):
            pltpu.async_copy(sbuf.at[k], zz.at[six.at[k]], sems, add=True)
        for k in range(nch_t):
            pltpu.make_async_copy(
                sbuf.at[k], zz.at[pl.ds(0, CH)], sems).wait()
        plsc.subcore_barrier()
        pltpu.sync_copy(zz.at[pl.ds(sid * zlen, zlen)],
                        agg_hbm.at[cid, pl.ds(sid * zlen, zlen)])

    return sc_edge


def _make_sc_hgather(n, s_pad):
    nch_s = s_pad // (NW * CH)

    @functools.partial(
        pl.kernel,
        out_type=jax.ShapeDtypeStruct((s_pad, EMB), f32),
        mesh=_mesh(),
        compiler_params=_SC_PARAMS,
        scratch_types=[
            pltpu.VMEM((nch_s, CH), i32),
            pltpu.VMEM((nch_s * CH, EMB), f32),
            pltpu.SemaphoreType.DMA,
        ],
    )
    def sc_hgather(h_hbm, sidx_hbm, out_hbm, idxb, rowb, sem):
        w = _wid()
        pltpu.sync_copy(sidx_hbm.at[pl.ds(w * nch_s, nch_s)], idxb)
        for k in range(nch_s):
            pltpu.async_copy(h_hbm.at[idxb.at[k]],
                             rowb.at[pl.ds(k * CH, CH)], sem)
        for k in range(nch_s):
            pltpu.make_async_copy(h_hbm.at[pl.ds(0, CH)],
                                  rowb.at[pl.ds(k * CH, CH)], sem).wait()
        pltpu.sync_copy(rowb, out_hbm.at[pl.ds(w * nch_s * CH, nch_s * CH)])

    return sc_hgather


def kernel(x, edge_index, edge_attrs, stem_types, stems, stems_batch, batch,
           x_slices, frag_emb_w, stem_emb_w, bond_emb_w, conv_root_w,
           conv_bias, gru_w_ih, gru_w_hh, gru_b_ih, gru_b_hh, f2e_w1, f2e_b1,
           f2e_w2, f2e_b2, s2p_w1, s2p_b1, s2p_w2, s2p_b2, s2p_w3, s2p_b3,
           g2p_w1, g2p_b1, g2p_w2, g2p_b2):
    n = x.shape[0]
    e = edge_index.shape[1]
    s = stem_types.shape[0]
    bn = x_slices.shape[0] - 1
    n_bond = bond_emb_w.shape[0]
    npg = n // bn

    e_pad = _ceil_to(e, NW * CH)
    n_pad = _ceil_to(n, NW * CH)
    s_pad = _ceil_to(s, NW * CH)
    n_agg = _ceil_to(n + 16, NS * 8)

    srcp = jnp.concatenate([edge_index[0], jnp.zeros((e_pad - e,), i32)])
    dstp = jnp.concatenate([edge_index[1], jnp.full((e_pad - e,), n, i32)])
    e0p = jnp.concatenate([edge_attrs[:, 0], jnp.zeros((e_pad - e,), i32)])
    e1p = jnp.concatenate([edge_attrs[:, 1], jnp.zeros((e_pad - e,), i32)])
    xgp = jnp.concatenate([x, jnp.zeros((n_pad - n,), i32)])
    stgp = jnp.concatenate([stem_types, jnp.zeros((s_pad - s,), i32)])
    sbp = jnp.concatenate([stems_batch, jnp.zeros((s_pad - s,), i32)])
    s0p = jnp.concatenate([stems[:, 0], jnp.zeros((s_pad - s,), i32)])
    xslp = jnp.concatenate(
        [x_slices, jnp.zeros((256 - x_slices.shape[0],), i32)]).reshape(16, 16)
    xslp = lax.bitcast_convert_type(xslp, f32)
    zeros_z = jnp.zeros((_ceil_to(n_agg * n_bond, NS * 8),), f32)
    zeros_deg = jnp.zeros(((n + 16) // 16, 16), f32)

    w1at = s2p_w1[:, :EMB].T
    w1bt = s2p_w1[:, EMB:].T
    wirt, wizt, wint = (gru_w_ih[0:EMB].T, gru_w_ih[EMB:2 * EMB].T,
                        gru_w_ih[2 * EMB:].T)
    whrt, whzt, whnt = (gru_w_hh[0:EMB].T, gru_w_hh[EMB:2 * EMB].T,
                        gru_w_hh[2 * EMB:].T)
    bir, biz, bin_ = (gru_b_ih[0:EMB].reshape(1, EMB),
                      gru_b_ih[EMB:2 * EMB].reshape(1, EMB),
                      gru_b_ih[2 * EMB:].reshape(1, EMB))
    bhr, bhz, bhn = (gru_b_hh[0:EMB].reshape(1, EMB),
                     gru_b_hh[EMB:2 * EMB].reshape(1, EMB),
                     gru_b_hh[2 * EMB:].reshape(1, EMB))

    nf = frag_emb_w.shape[0]
    nst = stem_emb_w.shape[0]
    t1, t2 = pl.pallas_call(
        _tc_prep_body,
        out_shape=(jax.ShapeDtypeStruct((nf, EMB), f32),
                   jax.ShapeDtypeStruct((nst, EMB), f32)),
    )(frag_emb_w, f2e_w1.T, f2e_b1.reshape(1, EMB), f2e_w2.T,
      f2e_b2.reshape(1, EMB), stem_emb_w, w1bt)

    sc_prep = _make_sc_prep(n_pad, s_pad, n, e_pad, n_bond)
    h0p, st2p, sidxp, degpart, gidx2, zidx2 = sc_prep(
        t1, t2, xgp.reshape(-1, CH), stgp.reshape(-1, CH),
        sbp.reshape(-1, CH), s0p.reshape(-1, CH), xslp,
        srcp.reshape(-1, CH), dstp.reshape(-1, CH),
        e0p.reshape(-1, CH), e1p.reshape(-1, CH), zeros_deg)
    h = h0p[:n]

    sc_edge = _make_sc_edge(n, n_agg, e_pad, n_bond)
    rb = 1000
    grid = n // rb
    bondt = bond_emb_w.T
    p = pl.pallas_call(
        _tc_p0_body,
        grid=(grid,),
        in_specs=[
            pl.BlockSpec((rb, EMB), lambda i: (i, 0)),
            pl.BlockSpec((EMB, n_bond), lambda i: (0, 0)),
        ],
        out_specs=pl.BlockSpec((rb, n_bond), lambda i: (i, 0)),
        out_shape=jax.ShapeDtypeStruct((n, n_bond), f32),
    )(h, bondt)
    gru_call = pl.pallas_call(
        _tc_gru_body,
        grid=(grid,),
        in_specs=[
            pl.BlockSpec((NC, rb, n_bond), lambda i: (0, i, 0)),
            pl.BlockSpec((rb, EMB), lambda i: (i, 0)),
            pl.BlockSpec((rb, NW), lambda i: (i, 0)),
            pl.BlockSpec((n_bond, EMB), lambda i: (0, 0)),
            pl.BlockSpec((EMB, n_bond), lambda i: (0, 0)),
        ] + [pl.BlockSpec((EMB, EMB), lambda i: (0, 0))] * 7
          + [pl.BlockSpec((1, EMB), lambda i: (0, 0))] * 7,
        out_specs=[pl.BlockSpec((rb, EMB), lambda i: (i, 0)),
                   pl.BlockSpec((rb, n_bond), lambda i: (i, 0))],
        out_shape=(jax.ShapeDtypeStruct((n, EMB), f32),
                   jax.ShapeDtypeStruct((n, n_bond), f32)),
    )
    degpt = degpart.reshape(NW, -1)[:, :n].T
    for _ in range(NUM_CONV_STEPS):
        aggz = sc_edge(p.reshape(-1), gidx2, zidx2, zeros_z)
        aggz = aggz.reshape(NC, n_agg, n_bond)
        h, p = gru_call(aggz, h, degpt, bond_emb_w, bondt,
                        conv_root_w.T, wirt, wizt, wint,
                        whrt, whzt, whnt, conv_bias.reshape(1, EMB),
                        bir, biz, bin_, bhr, bhz, bhn)

    sc_hgather = _make_sc_hgather(n, s_pad)
    hsp = sc_hgather(h, sidxp)

    ops = s2p_w3.shape[0]
    sb_rows = 2000
    stem_preds = pl.pallas_call(
        _tc_stems_body,
        grid=(s // sb_rows,),
        in_specs=[
            pl.BlockSpec((sb_rows, EMB), lambda i: (i, 0)),
            pl.BlockSpec((sb_rows, EMB), lambda i: (i, 0)),
            pl.BlockSpec((EMB, EMB), lambda i: (0, 0)),
            pl.BlockSpec((1, EMB), lambda i: (0, 0)),
            pl.BlockSpec((EMB, EMB), lambda i: (0, 0)),
            pl.BlockSpec((1, EMB), lambda i: (0, 0)),
            pl.BlockSpec((EMB, ops), lambda i: (0, 0)),
            pl.BlockSpec((1, ops), lambda i: (0, 0)),
        ],
        out_specs=pl.BlockSpec((sb_rows, ops), lambda i: (i, 0)),
        out_shape=jax.ShapeDtypeStruct((s, ops), f32),
    )(hsp[:s], st2p[:s], w1at, s2p_b1.reshape(1, EMB), s2p_w2.T,
      s2p_b2.reshape(1, EMB), s2p_w3.T, s2p_b3.reshape(1, ops))

    opm = g2p_w2.shape[0]
    mol_preds = pl.pallas_call(
        _tc_pool_body,
        out_shape=jax.ShapeDtypeStruct((bn, opm), f32),
    )(h.reshape(bn, npg, EMB), g2p_w1.T, g2p_b1.reshape(1, EMB), g2p_w2.T,
      g2p_b2.reshape(1, opm))

    return (mol_preds, stem_preds)
